# Initial kernel scaffold; baseline (speedup 1.0000x reference)
import jax
import jax.numpy as jnp
from jax.experimental import pallas as pl


def kernel(user, food, edge_index, Wu, bu, Wf, bf, Wq, Wk, pool_w, user_table, item_table):
    raise NotImplementedError("write your pallas kernel here")


# speedup vs baseline: 8.2821x; 8.2821x over previous
import functools

import jax
import jax.numpy as jnp
from jax import lax
from jax.experimental import pallas as pl
from jax.experimental.pallas import tpu as pltpu
from jax.experimental.pallas import tpu_sc as plsc

NUM_USERS = 25000
NUM_FOODS = 25000
NR = NUM_USERS + NUM_FOODS
D_IN = 128
D_EMB = 64
N_HEADS = 4
HEAD_DIM = 16
THRESH = 0.3
E = 800000

NC = 2
NS = 16
NP = 49 * 1024
APAD = 25088
EP = 2 * 16 * 25088
E_PER_SC = EP // NC
CH = 128
ROWB = 1024

_mesh = plsc.VectorSubcoreMesh(
    core_axis_name="c", subcore_axis_name="s", num_cores=NC, num_subcores=NS)


def _proj_kernel_body(nblk_user, x_ref, w_ref, b_ref, wqk_ref, o_ref):
    pid = pl.program_id(0)
    is_user = pid < nblk_user
    w = jnp.where(is_user, w_ref[0], w_ref[1])
    b = jnp.where(is_user, b_ref[0], b_ref[1])
    wqk = jnp.where(is_user, wqk_ref[0], wqk_ref[1])
    emb = jnp.maximum(
        jnp.dot(x_ref[...], w, preferred_element_type=jnp.float32) + b[None, :],
        0.0)
    y = jnp.dot(emb, wqk, preferred_element_type=jnp.float32)
    lane = lax.broadcasted_iota(jnp.int32, y.shape, 1)
    scale = jnp.zeros_like(y)
    for h in range(N_HEADS):
        m = (lane >= h * HEAD_DIM) & (lane < (h + 1) * HEAD_DIM)
        n2 = jnp.sum(jnp.where(m, y * y, 0.0), axis=1, keepdims=True)
        scale = scale + jnp.where(m, lax.rsqrt(n2 + 1e-16), 0.0)
    o_ref[...] = y * scale


def _make_qk(x, Wu, bu, Wf, bf, Wq, Wk):
    nb = NR // 1000
    wstk = jnp.stack([Wu, Wf])
    bstk = jnp.stack([bu, bf])
    qstk = jnp.stack([Wq, Wk])
    return pl.pallas_call(
        functools.partial(_proj_kernel_body, NUM_USERS // 1000),
        grid=(nb,),
        in_specs=[
            pl.BlockSpec((1000, D_IN), lambda i: (i, 0)),
            pl.BlockSpec((2, D_IN, D_EMB), lambda i: (0, 0, 0)),
            pl.BlockSpec((2, D_EMB), lambda i: (0, 0)),
            pl.BlockSpec((2, D_EMB, D_EMB), lambda i: (0, 0, 0)),
        ],
        out_specs=pl.BlockSpec((1000, D_EMB), lambda i: (i, 0)),
        out_shape=jax.ShapeDtypeStruct((NR, D_EMB), jnp.float32),
    )(x, wstk, bstk, qstk)


def _edge_mask_body(qk, srcg, dstg, c0v, c1v,
                    gidx, sidx, degp,
                    sv, dv, gfb, gub, sub, sfb, dub, ddb,
                    ubuf, fbuf, onesb, zb, consts, degacc, sem, sem2):
    c = lax.axis_index("c")
    s = lax.axis_index("s")
    pltpu.sync_copy(c0v, consts.at[0])
    pltpu.sync_copy(c1v, consts.at[1])

    def zloop(i, _):
        zb[pl.ds(i * 16, 16)] = jnp.zeros((16,), jnp.float32)
        return 0
    lax.fori_loop(0, 784 // 16, zloop, 0)
    def zcopy(i, _):
        pltpu.sync_copy(zb, degacc.at[pl.ds(s * 3136 + i * 784, 784)])
        return 0
    lax.fori_loop(0, 4, zcopy, 0)
    def oloop(i, _):
        onesb[pl.ds(i * 16, 16)] = jnp.ones((16,), jnp.float32)
        return 0
    lax.fori_loop(0, CH // 16, oloop, 0)
    plsc.subcore_barrier()

    c0 = consts[0]
    c1 = consts[1]
    wbase = (c * NS + s) * (EP // (NC * NS))

    def chunk(k, _):
        base = wbase + k * CH
        pltpu.sync_copy(srcg.at[pl.ds(base, CH)], sv)
        pltpu.sync_copy(dstg.at[pl.ds(base, CH)], dv)
        cp1 = pltpu.async_copy(qk.at[sv], ubuf, sem)
        cp2 = pltpu.async_copy(qk.at[dv], fbuf, sem2)
        cp1.wait()
        cp2.wait()
        for g in range(CH // 16):
            rows = jnp.full((16,), g * 16, jnp.int32) + lax.iota(jnp.int32, 16)
            acc = jnp.zeros((16,), jnp.float32)
            for d in range(D_EMB):
                cols = jnp.full((16,), d, jnp.int32)
                uv = plsc.load_gather(ubuf, [rows, cols])
                fv = plsc.load_gather(fbuf, [rows, cols])
                acc = acc + uv * fv
            eid = base + rows
            w = jnp.where(acc * 0.25 > THRESH, c1, c0)
            act = (w > 0.5) & (eid < E)
            sval = sv[pl.ds(g * 16, 16)]
            dval = dv[pl.ds(g * 16, 16)]
            tpad = NR + (eid & 127)
            apad = NUM_USERS + (eid & 63)
            dpad = NR + (eid & 127)
            gfb[pl.ds(g * 16, 16)] = jnp.where(act, dval, tpad)
            gub[pl.ds(g * 16, 16)] = jnp.where(act, sval, tpad)
            sub[pl.ds(g * 16, 16)] = jnp.where(act, sval, apad)
            sfb[pl.ds(g * 16, 16)] = jnp.where(act, dval - NUM_USERS, apad)
            dub[pl.ds(g * 16, 16)] = jnp.where(act, sval, dpad)
            ddb[pl.ds(g * 16, 16)] = jnp.where(act, dval, dpad)
        pltpu.sync_copy(gfb, gidx.at[pl.ds(base, CH)])
        pltpu.sync_copy(gub, gidx.at[pl.ds(EP + base, CH)])
        pltpu.sync_copy(sub, sidx.at[pl.ds(base, CH)])
        pltpu.sync_copy(sfb, sidx.at[pl.ds(EP + base, CH)])
        pltpu.sync_copy(onesb, degacc.at[dub], add=True)
        pltpu.sync_copy(onesb, degacc.at[ddb], add=True)
        return 0

    lax.fori_loop(0, EP // (NC * NS) // CH, chunk, 0)
    plsc.subcore_barrier()
    pltpu.sync_copy(degacc.at[pl.ds(s * 3136, 3136)],
                    degp.at[pl.ds(c * NP + s * 3136, 3136)])


def _edge_mask(qk, srcg, dstg, c0v, c1v):
    f32 = jnp.float32
    i32 = jnp.int32
    return pl.kernel(
        _edge_mask_body,
        out_type=[
            jax.ShapeDtypeStruct((2 * EP,), i32),
            jax.ShapeDtypeStruct((2 * EP,), i32),
            jax.ShapeDtypeStruct((2 * NP,), f32),
        ],
        mesh=_mesh,
        compiler_params=pltpu.CompilerParams(
            needs_layout_passes=False, use_tc_tiling_on_sc=False),
        scratch_types=[
            pltpu.VMEM((CH,), i32),
            pltpu.VMEM((CH,), i32),
            pltpu.VMEM((CH,), i32),
            pltpu.VMEM((CH,), i32),
            pltpu.VMEM((CH,), i32),
            pltpu.VMEM((CH,), i32),
            pltpu.VMEM((CH,), i32),
            pltpu.VMEM((CH,), i32),
            pltpu.VMEM((CH, D_EMB), f32),
            pltpu.VMEM((CH, D_EMB), f32),
            pltpu.VMEM((CH,), f32),
            pltpu.VMEM((784,), f32),
            pltpu.VMEM((2, 16), f32),
            pltpu.VMEM_SHARED((NP,), f32),
            pltpu.SemaphoreType.DMA,
            pltpu.SemaphoreType.DMA,
        ],
    )(qk, srcg, dstg, c0v, c1v)


def _scale_body(degt_ref, x0_ref, t0_ref, r_ref):
    deg = degt_ref[:, 0:1] + degt_ref[:, 1:2]
    r = lax.rsqrt(jnp.maximum(deg, 0.5))
    r_ref[...] = r
    t0_ref[...] = x0_ref[...] * r


def _make_scale(degt, x0p):
    return pl.pallas_call(
        _scale_body,
        grid=(NP // ROWB,),
        in_specs=[
            pl.BlockSpec((ROWB, 2), lambda i: (i, 0)),
            pl.BlockSpec((ROWB, D_EMB), lambda i: (i, 0)),
        ],
        out_specs=[
            pl.BlockSpec((ROWB, D_EMB), lambda i: (i, 0)),
            pl.BlockSpec((ROWB, 1), lambda i: (i, 0)),
        ],
        out_shape=[
            jax.ShapeDtypeStruct((NP, D_EMB), jnp.float32),
            jax.ShapeDtypeStruct((NP, 1), jnp.float32),
        ],
    )(degt, x0p)


def _spmm_body(t_tab, gidx, sidx, s2,
               giv, siv, gbuf, zb, accum, sem):
    c = lax.axis_index("c")
    s = lax.axis_index("s")
    def zloop(i, _):
        for q in range(4):
            zb[i, pl.ds(q * 16, 16)] = jnp.zeros((16,), jnp.float32)
        return 0
    lax.fori_loop(0, 112, zloop, 0)
    def zcopy(i, _):
        pltpu.sync_copy(zb, accum.at[pl.ds(s * 1568 + i * 112, 112), :])
        return 0
    lax.fori_loop(0, 14, zcopy, 0)
    plsc.subcore_barrier()

    wbase = s * (EP // NS)

    def chunk(k, _):
        base = wbase + k * CH
        pltpu.sync_copy(gidx.at[pl.ds(c * EP + base, CH)], giv)
        pltpu.sync_copy(sidx.at[pl.ds(c * EP + base, CH)], siv)
        pltpu.async_copy(t_tab.at[giv], gbuf, sem).wait()
        pltpu.sync_copy(gbuf, accum.at[siv], add=True)
        return 0

    lax.fori_loop(0, EP // NS // CH, chunk, 0)
    plsc.subcore_barrier()
    pltpu.sync_copy(accum.at[pl.ds(s * 1568, 1568), :],
                    s2.at[c, pl.ds(s * 1568, 1568), :])


def _spmm(t_tab, gidx, sidx):
    f32 = jnp.float32
    return pl.kernel(
        _spmm_body,
        out_type=[jax.ShapeDtypeStruct((2, APAD, D_EMB), f32)],
        mesh=_mesh,
        compiler_params=pltpu.CompilerParams(
            needs_layout_passes=False, use_tc_tiling_on_sc=False),
        scratch_types=[
            pltpu.VMEM((CH,), jnp.int32),
            pltpu.VMEM((CH,), jnp.int32),
            pltpu.VMEM((CH, D_EMB), f32),
            pltpu.VMEM((112, D_EMB), f32),
            pltpu.VMEM_SHARED((APAD, D_EMB), f32),
            pltpu.SemaphoreType.DMA,
        ],
    )(t_tab, gidx, sidx)[0]


def _layer_body(final, acc_ref, s_ref, r_ref, accn_ref, tn_ref):
    r = r_ref[...]
    h = s_ref[...] * r
    acc = acc_ref[...] + h
    if final:
        accn_ref[...] = acc * 0.25
    else:
        accn_ref[...] = acc
    tn_ref[...] = h * r


def _layer_update(acc, s, r, final):
    return pl.pallas_call(
        functools.partial(_layer_body, final),
        grid=(NP // ROWB,),
        in_specs=[
            pl.BlockSpec((ROWB, D_EMB), lambda i: (i, 0)),
            pl.BlockSpec((ROWB, D_EMB), lambda i: (i, 0)),
            pl.BlockSpec((ROWB, 1), lambda i: (i, 0)),
        ],
        out_specs=[
            pl.BlockSpec((ROWB, D_EMB), lambda i: (i, 0)),
            pl.BlockSpec((ROWB, D_EMB), lambda i: (i, 0)),
        ],
        out_shape=[
            jax.ShapeDtypeStruct((NP, D_EMB), jnp.float32),
            jax.ShapeDtypeStruct((NP, D_EMB), jnp.float32),
        ],
    )(acc, s, r)


def kernel(user, food, edge_index, Wu, bu, Wf, bf, Wq, Wk, pool_w,
           user_table, item_table):
    f32 = jnp.float32
    src = edge_index[0].astype(jnp.int32)
    dst = edge_index[1].astype(jnp.int32)
    srcg = jnp.pad(src, (0, EP - E))
    dstg = jnp.pad(dst, (0, EP - E), constant_values=NUM_USERS)

    alpha = jax.nn.softmax(pool_w)
    c0 = (alpha[0] > 0.5).astype(f32)
    c1 = (alpha[0] + alpha[1] > 0.5).astype(f32)
    c0v = jnp.full((16,), c0, f32)
    c1v = jnp.full((16,), c1, f32)

    x = jnp.concatenate([user, food], axis=0)
    qk = _make_qk(x, Wu, bu, Wf, bf, Wq, Wk)

    gidx, sidx, degp = _edge_mask(qk, srcg, dstg, c0v, c1v)

    degt = degp.reshape(2, NP).T
    x0 = jnp.concatenate(
        [user_table, item_table, jnp.zeros((NP - NR, D_EMB), f32)], axis=0)
    t_tab, r = _make_scale(degt, x0)

    acc = x0
    for layer in range(3):
        s2 = _spmm(t_tab, gidx, sidx)
        s_cat = jnp.concatenate(
            [s2[0, :NUM_USERS], s2[1, :NUM_FOODS],
             jnp.zeros((NP - NR, D_EMB), f32)], axis=0)
        acc, t_tab = _layer_update(acc, s_cat, r, final=(layer == 2))

    users_final = acc[:NUM_USERS]
    items_final = acc[NUM_USERS:NR]
    return users_final, items_final, user_table, item_table


# speedup vs baseline: 8.3807x; 1.0119x over previous
import functools

import jax
import jax.numpy as jnp
from jax import lax
from jax.experimental import pallas as pl
from jax.experimental.pallas import tpu as pltpu
from jax.experimental.pallas import tpu_sc as plsc

NUM_USERS = 25000
NUM_FOODS = 25000
NR = NUM_USERS + NUM_FOODS
D_IN = 128
D_EMB = 64
N_HEADS = 4
HEAD_DIM = 16
THRESH = 0.3
E = 800000

NC = 2
NS = 16
NP = 49 * 1024
APAD = 25088
EP = 2 * 16 * 25088
E_PER_SC = EP // NC
CH = 128
ROWB = 1024

_mesh = plsc.VectorSubcoreMesh(
    core_axis_name="c", subcore_axis_name="s", num_cores=NC, num_subcores=NS)


def _proj_kernel_body(nblk_user, x_ref, w_ref, b_ref, wqk_ref, o_ref):
    pid = pl.program_id(0)
    is_user = pid < nblk_user
    w = jnp.where(is_user, w_ref[0], w_ref[1])
    b = jnp.where(is_user, b_ref[0], b_ref[1])
    wqk = jnp.where(is_user, wqk_ref[0], wqk_ref[1])
    emb = jnp.maximum(
        jnp.dot(x_ref[...], w, preferred_element_type=jnp.float32) + b[None, :],
        0.0)
    y = jnp.dot(emb, wqk, preferred_element_type=jnp.float32)
    lane = lax.broadcasted_iota(jnp.int32, y.shape, 1)
    scale = jnp.zeros_like(y)
    for h in range(N_HEADS):
        m = (lane >= h * HEAD_DIM) & (lane < (h + 1) * HEAD_DIM)
        n2 = jnp.sum(jnp.where(m, y * y, 0.0), axis=1, keepdims=True)
        scale = scale + jnp.where(m, lax.rsqrt(n2 + 1e-16), 0.0)
    o_ref[...] = y * scale


def _make_qk(x, Wu, bu, Wf, bf, Wq, Wk):
    nb = NR // 1000
    wstk = jnp.stack([Wu, Wf])
    bstk = jnp.stack([bu, bf])
    qstk = jnp.stack([Wq, Wk])
    return pl.pallas_call(
        functools.partial(_proj_kernel_body, NUM_USERS // 1000),
        grid=(nb,),
        in_specs=[
            pl.BlockSpec((1000, D_IN), lambda i: (i, 0)),
            pl.BlockSpec((2, D_IN, D_EMB), lambda i: (0, 0, 0)),
            pl.BlockSpec((2, D_EMB), lambda i: (0, 0)),
            pl.BlockSpec((2, D_EMB, D_EMB), lambda i: (0, 0, 0)),
        ],
        out_specs=pl.BlockSpec((1000, D_EMB), lambda i: (i, 0)),
        out_shape=jax.ShapeDtypeStruct((NR, D_EMB), jnp.float32),
    )(x, wstk, bstk, qstk)


CB = 256
NKB = 25088 // CB


def _edge_mask_body(qk, srcg, dstg, c0v, c1v,
                    gidx, sidx, degp,
                    sv, dv, gfb, gub, sub, sfb, dub, ddb,
                    ubuf, fbuf, onesb, zb, consts, degacc, sem, sem2):
    c = lax.axis_index("c")
    s = lax.axis_index("s")
    pltpu.sync_copy(c0v, consts.at[0])
    pltpu.sync_copy(c1v, consts.at[1])

    def zloop(i, _):
        zb[pl.ds(i * 16, 16)] = jnp.zeros((16,), jnp.float32)
        return 0
    lax.fori_loop(0, 784 // 16, zloop, 0)
    def zcopy(i, _):
        pltpu.sync_copy(zb, degacc.at[pl.ds(s * 3136 + i * 784, 784)])
        return 0
    lax.fori_loop(0, 4, zcopy, 0)
    def oloop(i, _):
        onesb[pl.ds(i * 16, 16)] = jnp.ones((16,), jnp.float32)
        return 0
    lax.fori_loop(0, 128 // 16, oloop, 0)
    plsc.subcore_barrier()

    c0 = consts[0]
    c1 = consts[1]
    wbase = (c * NS + s) * 25088

    def chunk(k, _):
        base = wbase + k * 128
        pltpu.sync_copy(srcg.at[pl.ds(base, 128)], sv)
        pltpu.sync_copy(dstg.at[pl.ds(base, 128)], dv)
        cp1 = pltpu.async_copy(qk.at[sv], ubuf, sem)
        cp2 = pltpu.async_copy(qk.at[dv], fbuf, sem2)
        cp1.wait()
        cp2.wait()
        for g in range(128 // 16):
            rows = jnp.full((16,), g * 16, jnp.int32) + lax.iota(jnp.int32, 16)
            acc = jnp.zeros((16,), jnp.float32)
            for d in range(D_EMB):
                cols = jnp.full((16,), d, jnp.int32)
                uv = plsc.load_gather(ubuf, [rows, cols])
                fv = plsc.load_gather(fbuf, [rows, cols])
                acc = acc + uv * fv
            eid = base + rows
            w = jnp.where(acc * 0.25 > THRESH, c1, c0)
            act = (w > 0.5) & (eid < E)
            sval = sv[pl.ds(g * 16, 16)]
            dval = dv[pl.ds(g * 16, 16)]
            tpad = NR + (eid & 127)
            apad = NUM_USERS + (eid & 63)
            dpad = NR + (eid & 127)
            gfb[pl.ds(g * 16, 16)] = jnp.where(act, dval, tpad)
            gub[pl.ds(g * 16, 16)] = jnp.where(act, sval, tpad)
            sub[pl.ds(g * 16, 16)] = jnp.where(act, sval, apad)
            sfb[pl.ds(g * 16, 16)] = jnp.where(act, dval - NUM_USERS, apad)
            dub[pl.ds(g * 16, 16)] = jnp.where(act, sval, dpad)
            ddb[pl.ds(g * 16, 16)] = jnp.where(act, dval, dpad)
        pltpu.sync_copy(gfb, gidx.at[pl.ds(base, 128)])
        pltpu.sync_copy(gub, gidx.at[pl.ds(EP + base, 128)])
        pltpu.sync_copy(sub, sidx.at[pl.ds(base, 128)])
        pltpu.sync_copy(sfb, sidx.at[pl.ds(EP + base, 128)])
        pltpu.sync_copy(onesb, degacc.at[dub], add=True)
        pltpu.sync_copy(onesb, degacc.at[ddb], add=True)
        return 0

    lax.fori_loop(0, 25088 // 128, chunk, 0)
    plsc.subcore_barrier()
    pltpu.sync_copy(degacc.at[pl.ds(s * 3136, 3136)],
                    degp.at[pl.ds(c * NP + s * 3136, 3136)])


def _edge_mask(qk, srcg, dstg, c0v, c1v):
    f32 = jnp.float32
    i32 = jnp.int32
    return pl.kernel(
        _edge_mask_body,
        out_type=[
            jax.ShapeDtypeStruct((2 * EP,), i32),
            jax.ShapeDtypeStruct((2 * EP,), i32),
            jax.ShapeDtypeStruct((2 * NP,), f32),
        ],
        mesh=_mesh,
        compiler_params=pltpu.CompilerParams(
            needs_layout_passes=False, use_tc_tiling_on_sc=False),
        scratch_types=[
            pltpu.VMEM((128,), i32),
            pltpu.VMEM((128,), i32),
            pltpu.VMEM((128,), i32),
            pltpu.VMEM((128,), i32),
            pltpu.VMEM((128,), i32),
            pltpu.VMEM((128,), i32),
            pltpu.VMEM((128,), i32),
            pltpu.VMEM((128,), i32),
            pltpu.VMEM((128, D_EMB), f32),
            pltpu.VMEM((128, D_EMB), f32),
            pltpu.VMEM((128,), f32),
            pltpu.VMEM((784,), f32),
            pltpu.VMEM((2, 16), f32),
            pltpu.VMEM_SHARED((NP,), f32),
            pltpu.SemaphoreType.DMA,
            pltpu.SemaphoreType.DMA,
        ],
    )(qk, srcg, dstg, c0v, c1v)


def _scale_body(degt_ref, x0_ref, t0_ref, r_ref):
    deg = degt_ref[:, 0:1] + degt_ref[:, 1:2]
    r = lax.rsqrt(jnp.maximum(deg, 0.5))
    r_ref[...] = r
    t0_ref[...] = x0_ref[...] * r


def _make_scale(degt, x0p):
    return pl.pallas_call(
        _scale_body,
        grid=(NP // ROWB,),
        in_specs=[
            pl.BlockSpec((ROWB, 2), lambda i: (i, 0)),
            pl.BlockSpec((ROWB, D_EMB), lambda i: (i, 0)),
        ],
        out_specs=[
            pl.BlockSpec((ROWB, D_EMB), lambda i: (i, 0)),
            pl.BlockSpec((ROWB, 1), lambda i: (i, 0)),
        ],
        out_shape=[
            jax.ShapeDtypeStruct((NP, D_EMB), jnp.float32),
            jax.ShapeDtypeStruct((NP, 1), jnp.float32),
        ],
    )(degt, x0p)


CD = 128
NKD = 50176 // CD


def _spmm_body(t_tab, gidx, sidx, s2,
               giv, siv, siv2, gbuf, zb, accum,
               sem_i0, sem_i1, sem_g0, sem_g1, sem_s0, sem_s1):
    c = lax.axis_index("c")
    s = lax.axis_index("s")
    sem_i = (sem_i0, sem_i1)
    sem_g = (sem_g0, sem_g1)
    sem_s = (sem_s0, sem_s1)
    def zloop(i, _):
        for q in range(4):
            zb[i, pl.ds(q * 16, 16)] = jnp.zeros((16,), jnp.float32)
        return 0
    lax.fori_loop(0, 112, zloop, 0)
    def zcopy(i, _):
        pltpu.sync_copy(zb, accum.at[pl.ds(s * 1568 + i * 112, 112), :])
        return 0
    lax.fori_loop(0, 14, zcopy, 0)
    plsc.subcore_barrier()

    wbase = s * (EP // NS)

    def idx_cp(k, b):
        base = c * EP + wbase + k * CD
        return (pltpu.make_async_copy(gidx.at[pl.ds(base, CD)], giv.at[b, 0],
                                      sem_i[b]),
                pltpu.make_async_copy(sidx.at[pl.ds(base, CD)], siv.at[b],
                                      sem_i[b]))

    def gather_cps(b):
        return [pltpu.make_async_copy(
            t_tab.at[giv.at[b, j]],
            gbuf.at[b, pl.ds(j * 128, 128), :], sem_g[b])
            for j in range(CD // 128)]

    def scatter_cps(b):
        return [pltpu.make_async_copy(
            gbuf.at[b, pl.ds(j * 128, 128), :],
            accum.at[siv2.at[b, j]], sem_s[b])
            for j in range(CD // 128)]

    for cp in idx_cp(0, 0):
        cp.start()
    for cp in idx_cp(1, 1):
        cp.start()

    def half(k2, k, b):
        for cp in idx_cp(k, b):
            cp.wait()
        for q in range(CD // 16):
            siv2[b, q // 8, pl.ds((q % 8) * 16, 16)] = siv[b, pl.ds(q * 16, 16)]
        for cp in gather_cps(b):
            cp.start()
        for cp in gather_cps(b):
            cp.wait()
        for cp in scatter_cps(b):
            cp.start(add=True)
        for cp in scatter_cps(b):
            cp.wait()
        @pl.when(k2 < (NKD // 2) - 1)
        def _():
            for cp in idx_cp(k + 2, b):
                cp.start()

    def body(k2, _):
        half(k2, 2 * k2, 0)
        half(k2, 2 * k2 + 1, 1)
        return 0

    lax.fori_loop(0, NKD // 2, body, 0)
    plsc.subcore_barrier()
    pltpu.sync_copy(accum.at[pl.ds(s * 1568, 1568), :],
                    s2.at[c, pl.ds(s * 1568, 1568), :])


def _spmm(t_tab, gidx, sidx):
    f32 = jnp.float32
    return pl.kernel(
        _spmm_body,
        out_type=[jax.ShapeDtypeStruct((2, APAD, D_EMB), f32)],
        mesh=_mesh,
        compiler_params=pltpu.CompilerParams(
            needs_layout_passes=False, use_tc_tiling_on_sc=False),
        scratch_types=[
            pltpu.VMEM((2, CD // 128, 128), jnp.int32),
            pltpu.VMEM((2, CD), jnp.int32),
            pltpu.VMEM((2, CD // 128, 128), jnp.int32),
            pltpu.VMEM((2, CD, D_EMB), f32),
            pltpu.VMEM((112, D_EMB), f32),
            pltpu.VMEM_SHARED((APAD, D_EMB), f32),
            pltpu.SemaphoreType.DMA,
            pltpu.SemaphoreType.DMA,
            pltpu.SemaphoreType.DMA,
            pltpu.SemaphoreType.DMA,
            pltpu.SemaphoreType.DMA,
            pltpu.SemaphoreType.DMA,
        ],
    )(t_tab, gidx, sidx)[0]


def _layer_body(final, acc_ref, s_ref, r_ref, accn_ref, tn_ref):
    r = r_ref[...]
    h = s_ref[...] * r
    acc = acc_ref[...] + h
    if final:
        accn_ref[...] = acc * 0.25
    else:
        accn_ref[...] = acc
    tn_ref[...] = h * r


def _layer_update(acc, s, r, final):
    return pl.pallas_call(
        functools.partial(_layer_body, final),
        grid=(NP // ROWB,),
        in_specs=[
            pl.BlockSpec((ROWB, D_EMB), lambda i: (i, 0)),
            pl.BlockSpec((ROWB, D_EMB), lambda i: (i, 0)),
            pl.BlockSpec((ROWB, 1), lambda i: (i, 0)),
        ],
        out_specs=[
            pl.BlockSpec((ROWB, D_EMB), lambda i: (i, 0)),
            pl.BlockSpec((ROWB, D_EMB), lambda i: (i, 0)),
        ],
        out_shape=[
            jax.ShapeDtypeStruct((NP, D_EMB), jnp.float32),
            jax.ShapeDtypeStruct((NP, D_EMB), jnp.float32),
        ],
    )(acc, s, r)


def kernel(user, food, edge_index, Wu, bu, Wf, bf, Wq, Wk, pool_w,
           user_table, item_table):
    f32 = jnp.float32
    src = edge_index[0].astype(jnp.int32)
    dst = edge_index[1].astype(jnp.int32)
    srcg = jnp.pad(src, (0, EP - E))
    dstg = jnp.pad(dst, (0, EP - E), constant_values=NUM_USERS)

    alpha = jax.nn.softmax(pool_w)
    c0 = (alpha[0] > 0.5).astype(f32)
    c1 = (alpha[0] + alpha[1] > 0.5).astype(f32)
    c0v = jnp.full((16,), c0, f32)
    c1v = jnp.full((16,), c1, f32)

    x = jnp.concatenate([user, food], axis=0)
    qk = _make_qk(x, Wu, bu, Wf, bf, Wq, Wk)

    gidx, sidx, degp = _edge_mask(qk, srcg, dstg, c0v, c1v)

    degt = degp.reshape(2, NP).T
    x0 = jnp.concatenate(
        [user_table, item_table, jnp.zeros((NP - NR, D_EMB), f32)], axis=0)
    t_tab, r = _make_scale(degt, x0)

    acc = x0
    for layer in range(3):
        s2 = _spmm(t_tab, gidx, sidx)
        s_full = jnp.concatenate(
            [s2[0, :NUM_USERS], s2[1, :NUM_FOODS],
             jnp.zeros((NP - NR, D_EMB), f32)], axis=0)
        acc, t_tab = _layer_update(acc, s_full, r, final=(layer == 2))

    users_final = acc[:NUM_USERS]
    items_final = acc[NUM_USERS:NR]
    return users_final, items_final, user_table, item_table


# speedup vs baseline: 8.6909x; 1.0370x over previous
import functools

import jax
import jax.numpy as jnp
from jax import lax
from jax.experimental import pallas as pl
from jax.experimental.pallas import tpu as pltpu
from jax.experimental.pallas import tpu_sc as plsc

NUM_USERS = 25000
NUM_FOODS = 25000
NR = NUM_USERS + NUM_FOODS
D_IN = 128
D_EMB = 64
N_HEADS = 4
HEAD_DIM = 16
THRESH = 0.3
E = 800000

NC = 2
NS = 16
NP = 49 * 1024
APAD = 25088
EP = 2 * 16 * 25088
E_PER_SC = EP // NC
CH = 128
ROWB = 1024

_mesh = plsc.VectorSubcoreMesh(
    core_axis_name="c", subcore_axis_name="s", num_cores=NC, num_subcores=NS)


def _proj_kernel_body(nblk_user, x_ref, w_ref, b_ref, wqk_ref, o_ref):
    pid = pl.program_id(0)
    is_user = pid < nblk_user
    w = jnp.where(is_user, w_ref[0], w_ref[1])
    b = jnp.where(is_user, b_ref[0], b_ref[1])
    wqk = jnp.where(is_user, wqk_ref[0], wqk_ref[1])
    emb = jnp.maximum(
        jnp.dot(x_ref[...], w, preferred_element_type=jnp.float32) + b[None, :],
        0.0)
    y = jnp.dot(emb, wqk, preferred_element_type=jnp.float32)
    lane = lax.broadcasted_iota(jnp.int32, y.shape, 1)
    scale = jnp.zeros_like(y)
    for h in range(N_HEADS):
        m = (lane >= h * HEAD_DIM) & (lane < (h + 1) * HEAD_DIM)
        n2 = jnp.sum(jnp.where(m, y * y, 0.0), axis=1, keepdims=True)
        scale = scale + jnp.where(m, lax.rsqrt(n2 + 1e-16), 0.0)
    o_ref[...] = y * scale


def _make_qk(x, Wu, bu, Wf, bf, Wq, Wk):
    nb = NR // 1000
    wstk = jnp.stack([Wu, Wf])
    bstk = jnp.stack([bu, bf])
    qstk = jnp.stack([Wq, Wk])
    return pl.pallas_call(
        functools.partial(_proj_kernel_body, NUM_USERS // 1000),
        grid=(nb,),
        in_specs=[
            pl.BlockSpec((1000, D_IN), lambda i: (i, 0)),
            pl.BlockSpec((2, D_IN, D_EMB), lambda i: (0, 0, 0)),
            pl.BlockSpec((2, D_EMB), lambda i: (0, 0)),
            pl.BlockSpec((2, D_EMB, D_EMB), lambda i: (0, 0, 0)),
        ],
        out_specs=pl.BlockSpec((1000, D_EMB), lambda i: (i, 0)),
        out_shape=jax.ShapeDtypeStruct((NR, D_EMB), jnp.float32),
    )(x, wstk, bstk, qstk)


CB = 128
NKB = 25088 // CB


def _edge_mask_body(qk, srcg, dstg, c0v, c1v,
                    gidx, sidx, degp,
                    sv, dv, gfb, gub, sub, sfb, dub, ddb,
                    ubuf, fbuf, onesb, zb, consts, degacc,
                    sem_i0, sem_i1, sem_g0, sem_g1, sem_o0, sem_o1):
    c = lax.axis_index("c")
    s = lax.axis_index("s")
    sem_i = (sem_i0, sem_i1)
    sem_g = (sem_g0, sem_g1)
    sem_o = (sem_o0, sem_o1)
    pltpu.sync_copy(c0v, consts.at[0])
    pltpu.sync_copy(c1v, consts.at[1])

    def zloop(i, _):
        zb[pl.ds(i * 16, 16)] = jnp.zeros((16,), jnp.float32)
        return 0
    lax.fori_loop(0, 784 // 16, zloop, 0)
    def zcopy(i, _):
        pltpu.sync_copy(zb, degacc.at[pl.ds(s * 3136 + i * 784, 784)])
        return 0
    lax.fori_loop(0, 4, zcopy, 0)
    def oloop(i, _):
        onesb[pl.ds(i * 16, 16)] = jnp.ones((16,), jnp.float32)
        return 0
    lax.fori_loop(0, 128 // 16, oloop, 0)
    plsc.subcore_barrier()

    c0 = consts[0]
    c1 = consts[1]
    wbase = (c * NS + s) * 25088

    def idx_cp(k, b):
        base = wbase + k * CB
        return (pltpu.make_async_copy(srcg.at[pl.ds(base, CB)], sv.at[b],
                                      sem_i[b]),
                pltpu.make_async_copy(dstg.at[pl.ds(base, CB)], dv.at[b],
                                      sem_i[b]))

    def gather_cps(b):
        return (pltpu.make_async_copy(qk.at[sv.at[b]], ubuf.at[b], sem_g[b]),
                pltpu.make_async_copy(qk.at[dv.at[b]], fbuf.at[b], sem_g[b]))

    def out_cps(k, b):
        base = wbase + k * CB
        return (
            pltpu.make_async_copy(gfb.at[b], gidx.at[pl.ds(base, CB)],
                                  sem_o[b]),
            pltpu.make_async_copy(gub.at[b], gidx.at[pl.ds(EP + base, CB)],
                                  sem_o[b]),
            pltpu.make_async_copy(sub.at[b], sidx.at[pl.ds(base, CB)],
                                  sem_o[b]),
            pltpu.make_async_copy(sfb.at[b], sidx.at[pl.ds(EP + base, CB)],
                                  sem_o[b]),
        )

    for cp in idx_cp(0, 0):
        cp.start()
    for cp in idx_cp(0, 0):
        cp.wait()
    for cp in gather_cps(0):
        cp.start()
    for cp in idx_cp(1, 1):
        cp.start()

    def compute_chunk(k, b):
        base = wbase + k * CB

        def group(g, _):
            rows = g * 16 + lax.iota(jnp.int32, 16)
            acc = jnp.zeros((16,), jnp.float32)
            for d in range(D_EMB):
                cols = jnp.full((16,), d, jnp.int32)
                uv = plsc.load_gather(ubuf.at[b], [rows, cols])
                fv = plsc.load_gather(fbuf.at[b], [rows, cols])
                acc = acc + uv * fv
            eid = base + rows
            w = jnp.where(acc * 0.25 > THRESH, c1, c0)
            act = (w > 0.5) & (eid < E)
            sval = sv[b, pl.ds(g * 16, 16)]
            dval = dv[b, pl.ds(g * 16, 16)]
            tpad = NR + (eid & 127)
            apad = NUM_USERS + (eid & 63)
            gfb[b, pl.ds(g * 16, 16)] = jnp.where(act, dval, tpad)
            gub[b, pl.ds(g * 16, 16)] = jnp.where(act, sval, tpad)
            sub[b, pl.ds(g * 16, 16)] = jnp.where(act, sval, apad)
            sfb[b, pl.ds(g * 16, 16)] = jnp.where(act, dval - NUM_USERS, apad)
            dub[pl.ds(g * 16, 16)] = jnp.where(act, sval, NR + (eid & 127))
            ddb[pl.ds(g * 16, 16)] = jnp.where(act, dval, NR + (eid & 127))
            return 0

        lax.fori_loop(0, CB // 16, group, 0)

    def half(k2, k, b):
        nb = 1 - b
        for cp in gather_cps(b):
            cp.wait()
        compute_chunk(k, b)
        pltpu.sync_copy(onesb, degacc.at[dub], add=True)
        pltpu.sync_copy(onesb, degacc.at[ddb], add=True)
        @pl.when(k < NKB - 1)
        def _():
            for cp in idx_cp(k + 1, nb):
                cp.wait()
            for cp in gather_cps(nb):
                cp.start()
        @pl.when(k2 >= 1)
        def _():
            for cp in out_cps(k - 2, b):
                cp.wait()
        for cp in out_cps(k, b):
            cp.start()
        @pl.when(k2 < (NKB // 2) - 1)
        def _():
            for cp in idx_cp(k + 2, b):
                cp.start()

    def body(k2, _):
        half(k2, 2 * k2, 0)
        half(k2, 2 * k2 + 1, 1)
        return 0

    lax.fori_loop(0, NKB // 2, body, 0)
    for cp in out_cps(NKB - 2, 0):
        cp.wait()
    for cp in out_cps(NKB - 1, 1):
        cp.wait()
    plsc.subcore_barrier()
    pltpu.sync_copy(degacc.at[pl.ds(s * 3136, 3136)],
                    degp.at[pl.ds(c * NP + s * 3136, 3136)])


def _edge_mask(qk, srcg, dstg, c0v, c1v):
    f32 = jnp.float32
    i32 = jnp.int32
    return pl.kernel(
        _edge_mask_body,
        out_type=[
            jax.ShapeDtypeStruct((2 * EP,), i32),
            jax.ShapeDtypeStruct((2 * EP,), i32),
            jax.ShapeDtypeStruct((2 * NP,), f32),
        ],
        mesh=_mesh,
        compiler_params=pltpu.CompilerParams(
            needs_layout_passes=False, use_tc_tiling_on_sc=False),
        scratch_types=[
            pltpu.VMEM((2, CB), i32),
            pltpu.VMEM((2, CB), i32),
            pltpu.VMEM((2, CB), i32),
            pltpu.VMEM((2, CB), i32),
            pltpu.VMEM((2, CB), i32),
            pltpu.VMEM((2, CB), i32),
            pltpu.VMEM((CB,), i32),
            pltpu.VMEM((CB,), i32),
            pltpu.VMEM((2, CB, D_EMB), f32),
            pltpu.VMEM((2, CB, D_EMB), f32),
            pltpu.VMEM((CB,), f32),
            pltpu.VMEM((784,), f32),
            pltpu.VMEM((2, 16), f32),
            pltpu.VMEM_SHARED((NP,), f32),
            pltpu.SemaphoreType.DMA,
            pltpu.SemaphoreType.DMA,
            pltpu.SemaphoreType.DMA,
            pltpu.SemaphoreType.DMA,
            pltpu.SemaphoreType.DMA,
            pltpu.SemaphoreType.DMA,
        ],
    )(qk, srcg, dstg, c0v, c1v)


def _scale_body(degt_ref, x0_ref, t0_ref, r_ref):
    deg = degt_ref[:, 0:1] + degt_ref[:, 1:2]
    r = lax.rsqrt(jnp.maximum(deg, 0.5))
    r_ref[...] = r
    t0_ref[...] = x0_ref[...] * r


def _make_scale(degt, x0p):
    return pl.pallas_call(
        _scale_body,
        grid=(NP // ROWB,),
        in_specs=[
            pl.BlockSpec((ROWB, 2), lambda i: (i, 0)),
            pl.BlockSpec((ROWB, D_EMB), lambda i: (i, 0)),
        ],
        out_specs=[
            pl.BlockSpec((ROWB, D_EMB), lambda i: (i, 0)),
            pl.BlockSpec((ROWB, 1), lambda i: (i, 0)),
        ],
        out_shape=[
            jax.ShapeDtypeStruct((NP, D_EMB), jnp.float32),
            jax.ShapeDtypeStruct((NP, 1), jnp.float32),
        ],
    )(degt, x0p)


CD = 128
NKD = 50176 // CD


def _spmm_body(t_tab, gidx, sidx, s2,
               giv, siv, siv2, gbuf, zb, accum,
               sem_i0, sem_i1, sem_g0, sem_g1, sem_s0, sem_s1):
    c = lax.axis_index("c")
    s = lax.axis_index("s")
    sem_i = (sem_i0, sem_i1)
    sem_g = (sem_g0, sem_g1)
    sem_s = (sem_s0, sem_s1)
    def zloop(i, _):
        for q in range(4):
            zb[i, pl.ds(q * 16, 16)] = jnp.zeros((16,), jnp.float32)
        return 0
    lax.fori_loop(0, 112, zloop, 0)
    def zcopy(i, _):
        pltpu.sync_copy(zb, accum.at[pl.ds(s * 1568 + i * 112, 112), :])
        return 0
    lax.fori_loop(0, 14, zcopy, 0)
    plsc.subcore_barrier()

    wbase = s * (EP // NS)

    def idx_cp(k, b):
        base = c * EP + wbase + k * CD
        return (pltpu.make_async_copy(gidx.at[pl.ds(base, CD)], giv.at[b, 0],
                                      sem_i[b]),
                pltpu.make_async_copy(sidx.at[pl.ds(base, CD)], siv.at[b],
                                      sem_i[b]))

    def gather_cps(b):
        return [pltpu.make_async_copy(
            t_tab.at[giv.at[b, j]],
            gbuf.at[b, pl.ds(j * 128, 128), :], sem_g[b])
            for j in range(CD // 128)]

    def scatter_cps(b):
        return [pltpu.make_async_copy(
            gbuf.at[b, pl.ds(j * 128, 128), :],
            accum.at[siv2.at[b, j]], sem_s[b])
            for j in range(CD // 128)]

    for cp in idx_cp(0, 0):
        cp.start()
    for cp in idx_cp(1, 1):
        cp.start()

    def half(k2, k, b):
        for cp in idx_cp(k, b):
            cp.wait()
        for q in range(CD // 16):
            siv2[b, q // 8, pl.ds((q % 8) * 16, 16)] = siv[b, pl.ds(q * 16, 16)]
        for cp in gather_cps(b):
            cp.start()
        for cp in gather_cps(b):
            cp.wait()
        for cp in scatter_cps(b):
            cp.start(add=True)
        for cp in scatter_cps(b):
            cp.wait()
        @pl.when(k2 < (NKD // 2) - 1)
        def _():
            for cp in idx_cp(k + 2, b):
                cp.start()

    def body(k2, _):
        half(k2, 2 * k2, 0)
        half(k2, 2 * k2 + 1, 1)
        return 0

    lax.fori_loop(0, NKD // 2, body, 0)
    plsc.subcore_barrier()
    pltpu.sync_copy(accum.at[pl.ds(s * 1568, 1568), :],
                    s2.at[c, pl.ds(s * 1568, 1568), :])


def _spmm(t_tab, gidx, sidx):
    f32 = jnp.float32
    return pl.kernel(
        _spmm_body,
        out_type=[jax.ShapeDtypeStruct((2, APAD, D_EMB), f32)],
        mesh=_mesh,
        compiler_params=pltpu.CompilerParams(
            needs_layout_passes=False, use_tc_tiling_on_sc=False),
        scratch_types=[
            pltpu.VMEM((2, CD // 128, 128), jnp.int32),
            pltpu.VMEM((2, CD), jnp.int32),
            pltpu.VMEM((2, CD // 128, 128), jnp.int32),
            pltpu.VMEM((2, CD, D_EMB), f32),
            pltpu.VMEM((112, D_EMB), f32),
            pltpu.VMEM_SHARED((APAD, D_EMB), f32),
            pltpu.SemaphoreType.DMA,
            pltpu.SemaphoreType.DMA,
            pltpu.SemaphoreType.DMA,
            pltpu.SemaphoreType.DMA,
            pltpu.SemaphoreType.DMA,
            pltpu.SemaphoreType.DMA,
        ],
    )(t_tab, gidx, sidx)[0]


def _layer_body(final, acc_ref, s_ref, r_ref, accn_ref, tn_ref):
    r = r_ref[...]
    h = s_ref[...] * r
    acc = acc_ref[...] + h
    if final:
        accn_ref[...] = acc * 0.25
    else:
        accn_ref[...] = acc
    tn_ref[...] = h * r


def _layer_update(acc, s, r, final):
    return pl.pallas_call(
        functools.partial(_layer_body, final),
        grid=(NP // ROWB,),
        in_specs=[
            pl.BlockSpec((ROWB, D_EMB), lambda i: (i, 0)),
            pl.BlockSpec((ROWB, D_EMB), lambda i: (i, 0)),
            pl.BlockSpec((ROWB, 1), lambda i: (i, 0)),
        ],
        out_specs=[
            pl.BlockSpec((ROWB, D_EMB), lambda i: (i, 0)),
            pl.BlockSpec((ROWB, D_EMB), lambda i: (i, 0)),
        ],
        out_shape=[
            jax.ShapeDtypeStruct((NP, D_EMB), jnp.float32),
            jax.ShapeDtypeStruct((NP, D_EMB), jnp.float32),
        ],
    )(acc, s, r)


def kernel(user, food, edge_index, Wu, bu, Wf, bf, Wq, Wk, pool_w,
           user_table, item_table):
    f32 = jnp.float32
    src = edge_index[0].astype(jnp.int32)
    dst = edge_index[1].astype(jnp.int32)
    srcg = jnp.pad(src, (0, EP - E))
    dstg = jnp.pad(dst, (0, EP - E), constant_values=NUM_USERS)

    alpha = jax.nn.softmax(pool_w)
    c0 = (alpha[0] > 0.5).astype(f32)
    c1 = (alpha[0] + alpha[1] > 0.5).astype(f32)
    c0v = jnp.full((16,), c0, f32)
    c1v = jnp.full((16,), c1, f32)

    x = jnp.concatenate([user, food], axis=0)
    qk = _make_qk(x, Wu, bu, Wf, bf, Wq, Wk)

    gidx, sidx, degp = _edge_mask(qk, srcg, dstg, c0v, c1v)

    degt = degp.reshape(2, NP).T
    x0 = jnp.concatenate(
        [user_table, item_table, jnp.zeros((NP - NR, D_EMB), f32)], axis=0)
    t_tab, r = _make_scale(degt, x0)

    acc = x0
    for layer in range(3):
        s2 = _spmm(t_tab, gidx, sidx)
        s_full = jnp.concatenate(
            [s2[0, :NUM_USERS], s2[1, :NUM_FOODS],
             jnp.zeros((NP - NR, D_EMB), f32)], axis=0)
        acc, t_tab = _layer_update(acc, s_full, r, final=(layer == 2))

    users_final = acc[:NUM_USERS]
    items_final = acc[NUM_USERS:NR]
    return users_final, items_final, user_table, item_table


# speedup vs baseline: 10.5491x; 1.2138x over previous
import functools

import jax
import jax.numpy as jnp
from jax import lax
from jax.experimental import pallas as pl
from jax.experimental.pallas import tpu as pltpu
from jax.experimental.pallas import tpu_sc as plsc

NUM_USERS = 25000
NUM_FOODS = 25000
NR = NUM_USERS + NUM_FOODS
D_IN = 128
D_EMB = 64
N_HEADS = 4
HEAD_DIM = 16
THRESH = 0.3
E = 800000

NC = 2
NS = 16
NP = 49 * 1024
APAD = 25088
EP = 2 * 16 * 25088
E_PER_SC = EP // NC
CH = 128
ROWB = 1024

_mesh = plsc.VectorSubcoreMesh(
    core_axis_name="c", subcore_axis_name="s", num_cores=NC, num_subcores=NS)


def _proj_kernel_body(nblk_user, x_ref, w_ref, b_ref, wqk_ref, o_ref):
    pid = pl.program_id(0)
    is_user = pid < nblk_user
    w = jnp.where(is_user, w_ref[0], w_ref[1])
    b = jnp.where(is_user, b_ref[0], b_ref[1])
    wqk = jnp.where(is_user, wqk_ref[0], wqk_ref[1])
    emb = jnp.maximum(
        jnp.dot(x_ref[...], w, preferred_element_type=jnp.float32) + b[None, :],
        0.0)
    y = jnp.dot(emb, wqk, preferred_element_type=jnp.float32)
    lane = lax.broadcasted_iota(jnp.int32, y.shape, 1)
    scale = jnp.zeros_like(y)
    for h in range(N_HEADS):
        m = (lane >= h * HEAD_DIM) & (lane < (h + 1) * HEAD_DIM)
        n2 = jnp.sum(jnp.where(m, y * y, 0.0), axis=1, keepdims=True)
        scale = scale + jnp.where(m, lax.rsqrt(n2 + 1e-16), 0.0)
    o_ref[...] = y * scale


def _make_qk(x, Wu, bu, Wf, bf, Wq, Wk):
    nb = NR // 1000
    wstk = jnp.stack([Wu, Wf])
    bstk = jnp.stack([bu, bf])
    qstk = jnp.stack([Wq, Wk])
    return pl.pallas_call(
        functools.partial(_proj_kernel_body, NUM_USERS // 1000),
        grid=(nb,),
        in_specs=[
            pl.BlockSpec((1000, D_IN), lambda i: (i, 0)),
            pl.BlockSpec((2, D_IN, D_EMB), lambda i: (0, 0, 0)),
            pl.BlockSpec((2, D_EMB), lambda i: (0, 0)),
            pl.BlockSpec((2, D_EMB, D_EMB), lambda i: (0, 0, 0)),
        ],
        out_specs=pl.BlockSpec((1000, D_EMB), lambda i: (i, 0)),
        out_shape=jax.ShapeDtypeStruct((NR, D_EMB), jnp.float32),
    )(x, wstk, bstk, qstk)


CB = 128
NKB = 25088 // CB


def _edge_mask_body(qk, srcg, dstg, c0v, c1v,
                    gidx, sidx, degp,
                    sv, dv, gfb, gub, sub, sfb, dub, ddb,
                    ubuf, fbuf, onesb, zb, consts, pT, degacc,
                    sem_i0, sem_i1, sem_g0, sem_g1, sem_o0, sem_o1):
    c = lax.axis_index("c")
    s = lax.axis_index("s")
    sem_i = (sem_i0, sem_i1)
    sem_g = (sem_g0, sem_g1)
    sem_o = (sem_o0, sem_o1)
    pltpu.sync_copy(c0v, consts.at[0])
    pltpu.sync_copy(c1v, consts.at[1])

    def zloop(i, _):
        zb[pl.ds(i * 16, 16)] = jnp.zeros((16,), jnp.float32)
        return 0
    lax.fori_loop(0, 784 // 16, zloop, 0)
    def zcopy(i, _):
        pltpu.sync_copy(zb, degacc.at[pl.ds(s * 3136 + i * 784, 784)])
        return 0
    lax.fori_loop(0, 4, zcopy, 0)
    def oloop(i, _):
        onesb[pl.ds(i * 16, 16)] = jnp.ones((16,), jnp.float32)
        return 0
    lax.fori_loop(0, 128 // 16, oloop, 0)
    plsc.subcore_barrier()

    c0 = consts[0]
    c1 = consts[1]
    wbase = (c * NS + s) * 25088

    def idx_cp(k, b):
        base = wbase + k * CB
        return (pltpu.make_async_copy(srcg.at[pl.ds(base, CB)], sv.at[b],
                                      sem_i[b]),
                pltpu.make_async_copy(dstg.at[pl.ds(base, CB)], dv.at[b],
                                      sem_i[b]))

    def gather_cps(b):
        return (pltpu.make_async_copy(qk.at[sv.at[b]], ubuf.at[b], sem_g[b]),
                pltpu.make_async_copy(qk.at[dv.at[b]], fbuf.at[b], sem_g[b]))

    def out_cps(k, b):
        base = wbase + k * CB
        return (
            pltpu.make_async_copy(gfb.at[b], gidx.at[pl.ds(base, CB)],
                                  sem_o[b]),
            pltpu.make_async_copy(gub.at[b], gidx.at[pl.ds(EP + base, CB)],
                                  sem_o[b]),
            pltpu.make_async_copy(sub.at[b], sidx.at[pl.ds(base, CB)],
                                  sem_o[b]),
            pltpu.make_async_copy(sfb.at[b], sidx.at[pl.ds(EP + base, CB)],
                                  sem_o[b]),
        )

    for cp in idx_cp(0, 0):
        cp.start()
    for cp in idx_cp(0, 0):
        cp.wait()
    for cp in gather_cps(0):
        cp.start()
    for cp in idx_cp(1, 1):
        cp.start()

    def compute_chunk(k, b):
        base = wbase + k * CB
        lane = lax.iota(jnp.int32, 16)
        lane129 = lane * 129

        def edge(e, _):
            for u in range(2):
                ee = e * 2 + u
                for q in range(4):
                    uv = ubuf[b, ee, pl.ds(q * 16, 16)]
                    fv = fbuf[b, ee, pl.ds(q * 16, 16)]
                    plsc.store_scatter(pT, [q * 2064 + lane129 + ee], uv * fv)
            return 0

        lax.fori_loop(0, CB // 2, edge, 0)

        def group(g, _):
            rows = g * 16 + lax.iota(jnp.int32, 16)
            acc = jnp.zeros((16,), jnp.float32)
            for d in range(D_EMB):
                acc = acc + pT[pl.ds(d * 129 + g * 16, 16)]
            eid = base + rows
            w = jnp.where(acc * 0.25 > THRESH, c1, c0)
            act = (w > 0.5) & (eid < E)
            sval = sv[b, pl.ds(g * 16, 16)]
            dval = dv[b, pl.ds(g * 16, 16)]
            tpad = NR + (eid & 127)
            apad = NUM_USERS + (eid & 63)
            gfb[b, pl.ds(g * 16, 16)] = jnp.where(act, dval, tpad)
            gub[b, pl.ds(g * 16, 16)] = jnp.where(act, sval, tpad)
            sub[b, pl.ds(g * 16, 16)] = jnp.where(act, sval, apad)
            sfb[b, pl.ds(g * 16, 16)] = jnp.where(act, dval - NUM_USERS, apad)
            dub[pl.ds(g * 16, 16)] = jnp.where(act, sval, NR + (eid & 127))
            ddb[pl.ds(g * 16, 16)] = jnp.where(act, dval, NR + (eid & 127))
            return 0

        lax.fori_loop(0, CB // 16, group, 0)

    def half(k2, k, b):
        nb = 1 - b
        for cp in gather_cps(b):
            cp.wait()
        compute_chunk(k, b)
        pltpu.sync_copy(onesb, degacc.at[dub], add=True)
        pltpu.sync_copy(onesb, degacc.at[ddb], add=True)
        @pl.when(k < NKB - 1)
        def _():
            for cp in idx_cp(k + 1, nb):
                cp.wait()
            for cp in gather_cps(nb):
                cp.start()
        @pl.when(k2 >= 1)
        def _():
            for cp in out_cps(k - 2, b):
                cp.wait()
        for cp in out_cps(k, b):
            cp.start()
        @pl.when(k2 < (NKB // 2) - 1)
        def _():
            for cp in idx_cp(k + 2, b):
                cp.start()

    def body(k2, _):
        half(k2, 2 * k2, 0)
        half(k2, 2 * k2 + 1, 1)
        return 0

    lax.fori_loop(0, NKB // 2, body, 0)
    for cp in out_cps(NKB - 2, 0):
        cp.wait()
    for cp in out_cps(NKB - 1, 1):
        cp.wait()
    plsc.subcore_barrier()
    pltpu.sync_copy(degacc.at[pl.ds(s * 3136, 3136)],
                    degp.at[pl.ds(c * NP + s * 3136, 3136)])


def _edge_mask(qk, srcg, dstg, c0v, c1v):
    f32 = jnp.float32
    i32 = jnp.int32
    return pl.kernel(
        _edge_mask_body,
        out_type=[
            jax.ShapeDtypeStruct((2 * EP,), i32),
            jax.ShapeDtypeStruct((2 * EP,), i32),
            jax.ShapeDtypeStruct((2 * NP,), f32),
        ],
        mesh=_mesh,
        compiler_params=pltpu.CompilerParams(
            needs_layout_passes=False, use_tc_tiling_on_sc=False),
        scratch_types=[
            pltpu.VMEM((2, CB), i32),
            pltpu.VMEM((2, CB), i32),
            pltpu.VMEM((2, CB), i32),
            pltpu.VMEM((2, CB), i32),
            pltpu.VMEM((2, CB), i32),
            pltpu.VMEM((2, CB), i32),
            pltpu.VMEM((CB,), i32),
            pltpu.VMEM((CB,), i32),
            pltpu.VMEM((2, CB, D_EMB), f32),
            pltpu.VMEM((2, CB, D_EMB), f32),
            pltpu.VMEM((CB,), f32),
            pltpu.VMEM((784,), f32),
            pltpu.VMEM((2, 16), f32),
            pltpu.VMEM((8256,), f32),
            pltpu.VMEM_SHARED((NP,), f32),
            pltpu.SemaphoreType.DMA,
            pltpu.SemaphoreType.DMA,
            pltpu.SemaphoreType.DMA,
            pltpu.SemaphoreType.DMA,
            pltpu.SemaphoreType.DMA,
            pltpu.SemaphoreType.DMA,
        ],
    )(qk, srcg, dstg, c0v, c1v)


def _scale_body(degt_ref, x0_ref, t0_ref, r_ref):
    deg = degt_ref[:, 0:1] + degt_ref[:, 1:2]
    r = lax.rsqrt(jnp.maximum(deg, 0.5))
    r_ref[...] = r
    t0_ref[...] = x0_ref[...] * r


def _make_scale(degt, x0p):
    return pl.pallas_call(
        _scale_body,
        grid=(NP // ROWB,),
        in_specs=[
            pl.BlockSpec((ROWB, 2), lambda i: (i, 0)),
            pl.BlockSpec((ROWB, D_EMB), lambda i: (i, 0)),
        ],
        out_specs=[
            pl.BlockSpec((ROWB, D_EMB), lambda i: (i, 0)),
            pl.BlockSpec((ROWB, 1), lambda i: (i, 0)),
        ],
        out_shape=[
            jax.ShapeDtypeStruct((NP, D_EMB), jnp.float32),
            jax.ShapeDtypeStruct((NP, 1), jnp.float32),
        ],
    )(degt, x0p)


CD = 128
NKD = 50176 // CD


def _spmm_body(t_tab, gidx, sidx, s2,
               giv, siv, siv2, gbuf, zb, accum,
               sem_i0, sem_i1, sem_g0, sem_g1, sem_s0, sem_s1):
    c = lax.axis_index("c")
    s = lax.axis_index("s")
    sem_i = (sem_i0, sem_i1)
    sem_g = (sem_g0, sem_g1)
    sem_s = (sem_s0, sem_s1)
    def zloop(i, _):
        for q in range(4):
            zb[i, pl.ds(q * 16, 16)] = jnp.zeros((16,), jnp.float32)
        return 0
    lax.fori_loop(0, 112, zloop, 0)
    def zcopy(i, _):
        pltpu.sync_copy(zb, accum.at[pl.ds(s * 1568 + i * 112, 112), :])
        return 0
    lax.fori_loop(0, 14, zcopy, 0)
    plsc.subcore_barrier()

    wbase = s * (EP // NS)

    def idx_cp(k, b):
        base = c * EP + wbase + k * CD
        return (pltpu.make_async_copy(gidx.at[pl.ds(base, CD)], giv.at[b, 0],
                                      sem_i[b]),
                pltpu.make_async_copy(sidx.at[pl.ds(base, CD)], siv.at[b],
                                      sem_i[b]))

    def gather_cps(b):
        return [pltpu.make_async_copy(
            t_tab.at[giv.at[b, j]],
            gbuf.at[b, pl.ds(j * 128, 128), :], sem_g[b])
            for j in range(CD // 128)]

    def scatter_cps(b):
        return [pltpu.make_async_copy(
            gbuf.at[b, pl.ds(j * 128, 128), :],
            accum.at[siv2.at[b, j]], sem_s[b])
            for j in range(CD // 128)]

    for cp in idx_cp(0, 0):
        cp.start()
    for cp in idx_cp(1, 1):
        cp.start()

    def half(k2, k, b):
        for cp in idx_cp(k, b):
            cp.wait()
        for q in range(CD // 16):
            siv2[b, q // 8, pl.ds((q % 8) * 16, 16)] = siv[b, pl.ds(q * 16, 16)]
        for cp in gather_cps(b):
            cp.start()
        for cp in gather_cps(b):
            cp.wait()
        for cp in scatter_cps(b):
            cp.start(add=True)
        for cp in scatter_cps(b):
            cp.wait()
        @pl.when(k2 < (NKD // 2) - 1)
        def _():
            for cp in idx_cp(k + 2, b):
                cp.start()

    def body(k2, _):
        half(k2, 2 * k2, 0)
        half(k2, 2 * k2 + 1, 1)
        return 0

    lax.fori_loop(0, NKD // 2, body, 0)
    plsc.subcore_barrier()
    pltpu.sync_copy(accum.at[pl.ds(s * 1568, 1568), :],
                    s2.at[c, pl.ds(s * 1568, 1568), :])


def _spmm(t_tab, gidx, sidx):
    f32 = jnp.float32
    return pl.kernel(
        _spmm_body,
        out_type=[jax.ShapeDtypeStruct((2, APAD, D_EMB), f32)],
        mesh=_mesh,
        compiler_params=pltpu.CompilerParams(
            needs_layout_passes=False, use_tc_tiling_on_sc=False),
        scratch_types=[
            pltpu.VMEM((2, CD // 128, 128), jnp.int32),
            pltpu.VMEM((2, CD), jnp.int32),
            pltpu.VMEM((2, CD // 128, 128), jnp.int32),
            pltpu.VMEM((2, CD, D_EMB), f32),
            pltpu.VMEM((112, D_EMB), f32),
            pltpu.VMEM_SHARED((APAD, D_EMB), f32),
            pltpu.SemaphoreType.DMA,
            pltpu.SemaphoreType.DMA,
            pltpu.SemaphoreType.DMA,
            pltpu.SemaphoreType.DMA,
            pltpu.SemaphoreType.DMA,
            pltpu.SemaphoreType.DMA,
        ],
    )(t_tab, gidx, sidx)[0]


def _layer_body(final, acc_ref, s_ref, r_ref, accn_ref, tn_ref):
    r = r_ref[...]
    h = s_ref[...] * r
    acc = acc_ref[...] + h
    if final:
        accn_ref[...] = acc * 0.25
    else:
        accn_ref[...] = acc
    tn_ref[...] = h * r


def _layer_update(acc, s, r, final):
    return pl.pallas_call(
        functools.partial(_layer_body, final),
        grid=(NP // ROWB,),
        in_specs=[
            pl.BlockSpec((ROWB, D_EMB), lambda i: (i, 0)),
            pl.BlockSpec((ROWB, D_EMB), lambda i: (i, 0)),
            pl.BlockSpec((ROWB, 1), lambda i: (i, 0)),
        ],
        out_specs=[
            pl.BlockSpec((ROWB, D_EMB), lambda i: (i, 0)),
            pl.BlockSpec((ROWB, D_EMB), lambda i: (i, 0)),
        ],
        out_shape=[
            jax.ShapeDtypeStruct((NP, D_EMB), jnp.float32),
            jax.ShapeDtypeStruct((NP, D_EMB), jnp.float32),
        ],
    )(acc, s, r)


def kernel(user, food, edge_index, Wu, bu, Wf, bf, Wq, Wk, pool_w,
           user_table, item_table):
    f32 = jnp.float32
    src = edge_index[0].astype(jnp.int32)
    dst = edge_index[1].astype(jnp.int32)
    srcg = jnp.pad(src, (0, EP - E))
    dstg = jnp.pad(dst, (0, EP - E), constant_values=NUM_USERS)

    alpha = jax.nn.softmax(pool_w)
    c0 = (alpha[0] > 0.5).astype(f32)
    c1 = (alpha[0] + alpha[1] > 0.5).astype(f32)
    c0v = jnp.full((16,), c0, f32)
    c1v = jnp.full((16,), c1, f32)

    x = jnp.concatenate([user, food], axis=0)
    qk = _make_qk(x, Wu, bu, Wf, bf, Wq, Wk)

    gidx, sidx, degp = _edge_mask(qk, srcg, dstg, c0v, c1v)

    degt = degp.reshape(2, NP).T
    x0 = jnp.concatenate(
        [user_table, item_table, jnp.zeros((NP - NR, D_EMB), f32)], axis=0)
    t_tab, r = _make_scale(degt, x0)

    acc = x0
    for layer in range(3):
        s2 = _spmm(t_tab, gidx, sidx)
        s_full = jnp.concatenate(
            [s2[0, :NUM_USERS], s2[1, :NUM_FOODS],
             jnp.zeros((NP - NR, D_EMB), f32)], axis=0)
        acc, t_tab = _layer_update(acc, s_full, r, final=(layer == 2))

    users_final = acc[:NUM_USERS]
    items_final = acc[NUM_USERS:NR]
    return users_final, items_final, user_table, item_table


# speedup vs baseline: 10.5589x; 1.0009x over previous
import functools

import jax
import jax.numpy as jnp
from jax import lax
from jax.experimental import pallas as pl
from jax.experimental.pallas import tpu as pltpu
from jax.experimental.pallas import tpu_sc as plsc

NUM_USERS = 25000
NUM_FOODS = 25000
NR = NUM_USERS + NUM_FOODS
D_IN = 128
D_EMB = 64
N_HEADS = 4
HEAD_DIM = 16
THRESH = 0.3
E = 800000

NC = 2
NS = 16
NP = 49 * 1024
APAD = 25088
EP = 2 * 16 * 25088
E_PER_SC = EP // NC
CH = 128
ROWB = 1024

_mesh = plsc.VectorSubcoreMesh(
    core_axis_name="c", subcore_axis_name="s", num_cores=NC, num_subcores=NS)


def _proj_kernel_body(nblk_user, x_ref, w_ref, b_ref, wqk_ref, o_ref):
    pid = pl.program_id(0)
    is_user = pid < nblk_user
    w = jnp.where(is_user, w_ref[0], w_ref[1])
    b = jnp.where(is_user, b_ref[0], b_ref[1])
    wqk = jnp.where(is_user, wqk_ref[0], wqk_ref[1])
    emb = jnp.maximum(
        jnp.dot(x_ref[...], w, preferred_element_type=jnp.float32) + b[None, :],
        0.0)
    y = jnp.dot(emb, wqk, preferred_element_type=jnp.float32)
    lane = lax.broadcasted_iota(jnp.int32, y.shape, 1)
    scale = jnp.zeros_like(y)
    for h in range(N_HEADS):
        m = (lane >= h * HEAD_DIM) & (lane < (h + 1) * HEAD_DIM)
        n2 = jnp.sum(jnp.where(m, y * y, 0.0), axis=1, keepdims=True)
        scale = scale + jnp.where(m, lax.rsqrt(n2 + 1e-16), 0.0)
    o_ref[...] = y * scale


def _make_qk(x, Wu, bu, Wf, bf, Wq, Wk):
    nb = NR // 1000
    wstk = jnp.stack([Wu, Wf])
    bstk = jnp.stack([bu, bf])
    qstk = jnp.stack([Wq, Wk])
    return pl.pallas_call(
        functools.partial(_proj_kernel_body, NUM_USERS // 1000),
        grid=(nb,),
        in_specs=[
            pl.BlockSpec((1000, D_IN), lambda i: (i, 0)),
            pl.BlockSpec((2, D_IN, D_EMB), lambda i: (0, 0, 0)),
            pl.BlockSpec((2, D_EMB), lambda i: (0, 0)),
            pl.BlockSpec((2, D_EMB, D_EMB), lambda i: (0, 0, 0)),
        ],
        out_specs=pl.BlockSpec((1000, D_EMB), lambda i: (i, 0)),
        out_shape=jax.ShapeDtypeStruct((NR, D_EMB), jnp.float32),
    )(x, wstk, bstk, qstk)


CB = 128
NKB = 25088 // CB


def _edge_mask_body(qk, srcg, dstg, c0v, c1v,
                    gidx, sidx, degp,
                    sv, dv, gfb, gub, sub, sfb, dub, ddb,
                    ubuf, fbuf, onesb, zb, consts, pT, degacc,
                    sem_i0, sem_i1, sem_g0, sem_g1, sem_o0, sem_o1):
    c = lax.axis_index("c")
    s = lax.axis_index("s")
    sem_i = (sem_i0, sem_i1)
    sem_g = (sem_g0, sem_g1)
    sem_o = (sem_o0, sem_o1)
    pltpu.sync_copy(c0v, consts.at[0])
    pltpu.sync_copy(c1v, consts.at[1])

    def zloop(i, _):
        zb[pl.ds(i * 16, 16)] = jnp.zeros((16,), jnp.float32)
        return 0
    lax.fori_loop(0, 784 // 16, zloop, 0)
    def zcopy(i, _):
        pltpu.sync_copy(zb, degacc.at[pl.ds(s * 3136 + i * 784, 784)])
        return 0
    lax.fori_loop(0, 4, zcopy, 0)
    def oloop(i, _):
        onesb[pl.ds(i * 16, 16)] = jnp.ones((16,), jnp.float32)
        return 0
    lax.fori_loop(0, 128 // 16, oloop, 0)
    plsc.subcore_barrier()

    c0 = consts[0]
    c1 = consts[1]
    wbase = (c * NS + s) * 25088

    def idx_cp(k, b):
        base = wbase + k * CB
        return (pltpu.make_async_copy(srcg.at[pl.ds(base, CB)], sv.at[b],
                                      sem_i[b]),
                pltpu.make_async_copy(dstg.at[pl.ds(base, CB)], dv.at[b],
                                      sem_i[b]))

    def gather_cps(b):
        return (pltpu.make_async_copy(qk.at[sv.at[b]], ubuf.at[b], sem_g[b]),
                pltpu.make_async_copy(qk.at[dv.at[b]], fbuf.at[b], sem_g[b]))

    def out_cps(k, b):
        base = wbase + k * CB
        return (
            pltpu.make_async_copy(gfb.at[b], gidx.at[pl.ds(base, CB)],
                                  sem_o[b]),
            pltpu.make_async_copy(gub.at[b], gidx.at[pl.ds(EP + base, CB)],
                                  sem_o[b]),
            pltpu.make_async_copy(sub.at[b], sidx.at[pl.ds(base, CB)],
                                  sem_o[b]),
            pltpu.make_async_copy(sfb.at[b], sidx.at[pl.ds(EP + base, CB)],
                                  sem_o[b]),
        )

    for cp in idx_cp(0, 0):
        cp.start()
    for cp in idx_cp(0, 0):
        cp.wait()
    for cp in gather_cps(0):
        cp.start()
    for cp in idx_cp(1, 1):
        cp.start()

    def compute_chunk(k, b):
        base = wbase + k * CB
        lane = lax.iota(jnp.int32, 16)
        lane129 = lane * 129

        def edge(e, _):
            for u in range(2):
                ee = e * 2 + u
                for q in range(4):
                    uv = ubuf[b, ee, pl.ds(q * 16, 16)]
                    fv = fbuf[b, ee, pl.ds(q * 16, 16)]
                    plsc.store_scatter(pT, [q * 2064 + lane129 + ee], uv * fv)
            return 0

        lax.fori_loop(0, CB // 2, edge, 0)

        def group(g, _):
            rows = g * 16 + lax.iota(jnp.int32, 16)
            acc = jnp.zeros((16,), jnp.float32)
            for d in range(D_EMB):
                acc = acc + pT[pl.ds(d * 129 + g * 16, 16)]
            eid = base + rows
            w = jnp.where(acc * 0.25 > THRESH, c1, c0)
            act = (w > 0.5) & (eid < E)
            sval = sv[b, pl.ds(g * 16, 16)]
            dval = dv[b, pl.ds(g * 16, 16)]
            tpad = NR + (eid & 127)
            apad = NUM_USERS + (eid & 63)
            gfb[b, pl.ds(g * 16, 16)] = jnp.where(act, dval, tpad)
            gub[b, pl.ds(g * 16, 16)] = jnp.where(act, sval, tpad)
            sub[b, pl.ds(g * 16, 16)] = jnp.where(act, sval, apad)
            sfb[b, pl.ds(g * 16, 16)] = jnp.where(act, dval - NUM_USERS, apad)
            dub[pl.ds(g * 16, 16)] = jnp.where(act, sval, NR + (eid & 127))
            ddb[pl.ds(g * 16, 16)] = jnp.where(act, dval, NR + (eid & 127))
            return 0

        lax.fori_loop(0, CB // 16, group, 0)

    def half(k2, k, b):
        nb = 1 - b
        for cp in gather_cps(b):
            cp.wait()
        compute_chunk(k, b)
        pltpu.sync_copy(onesb, degacc.at[dub], add=True)
        pltpu.sync_copy(onesb, degacc.at[ddb], add=True)
        @pl.when(k < NKB - 1)
        def _():
            for cp in idx_cp(k + 1, nb):
                cp.wait()
            for cp in gather_cps(nb):
                cp.start()
        @pl.when(k2 >= 1)
        def _():
            for cp in out_cps(k - 2, b):
                cp.wait()
        for cp in out_cps(k, b):
            cp.start()
        @pl.when(k2 < (NKB // 2) - 1)
        def _():
            for cp in idx_cp(k + 2, b):
                cp.start()

    def body(k2, _):
        half(k2, 2 * k2, 0)
        half(k2, 2 * k2 + 1, 1)
        return 0

    lax.fori_loop(0, NKB // 2, body, 0)
    for cp in out_cps(NKB - 2, 0):
        cp.wait()
    for cp in out_cps(NKB - 1, 1):
        cp.wait()
    plsc.subcore_barrier()
    pltpu.sync_copy(degacc.at[pl.ds(s * 3136, 3136)],
                    degp.at[pl.ds(c * NP + s * 3136, 3136)])


def _edge_mask(qk, srcg, dstg, c0v, c1v):
    f32 = jnp.float32
    i32 = jnp.int32
    return pl.kernel(
        _edge_mask_body,
        out_type=[
            jax.ShapeDtypeStruct((2 * EP,), i32),
            jax.ShapeDtypeStruct((2 * EP,), i32),
            jax.ShapeDtypeStruct((2 * NP,), f32),
        ],
        mesh=_mesh,
        compiler_params=pltpu.CompilerParams(
            needs_layout_passes=False, use_tc_tiling_on_sc=False),
        scratch_types=[
            pltpu.VMEM((2, CB), i32),
            pltpu.VMEM((2, CB), i32),
            pltpu.VMEM((2, CB), i32),
            pltpu.VMEM((2, CB), i32),
            pltpu.VMEM((2, CB), i32),
            pltpu.VMEM((2, CB), i32),
            pltpu.VMEM((CB,), i32),
            pltpu.VMEM((CB,), i32),
            pltpu.VMEM((2, CB, D_EMB), f32),
            pltpu.VMEM((2, CB, D_EMB), f32),
            pltpu.VMEM((CB,), f32),
            pltpu.VMEM((784,), f32),
            pltpu.VMEM((2, 16), f32),
            pltpu.VMEM((8256,), f32),
            pltpu.VMEM_SHARED((NP,), f32),
            pltpu.SemaphoreType.DMA,
            pltpu.SemaphoreType.DMA,
            pltpu.SemaphoreType.DMA,
            pltpu.SemaphoreType.DMA,
            pltpu.SemaphoreType.DMA,
            pltpu.SemaphoreType.DMA,
        ],
    )(qk, srcg, dstg, c0v, c1v)


def _scale_body(degt_ref, x0_ref, t0_ref, r_ref):
    deg = degt_ref[:, 0:1] + degt_ref[:, 1:2]
    r = lax.rsqrt(jnp.maximum(deg, 0.5))
    r_ref[...] = r
    t0_ref[...] = x0_ref[...] * r


def _make_scale(degt, x0p):
    return pl.pallas_call(
        _scale_body,
        grid=(NP // ROWB,),
        in_specs=[
            pl.BlockSpec((ROWB, 2), lambda i: (i, 0)),
            pl.BlockSpec((ROWB, D_EMB), lambda i: (i, 0)),
        ],
        out_specs=[
            pl.BlockSpec((ROWB, D_EMB), lambda i: (i, 0)),
            pl.BlockSpec((ROWB, 1), lambda i: (i, 0)),
        ],
        out_shape=[
            jax.ShapeDtypeStruct((NP, D_EMB), jnp.float32),
            jax.ShapeDtypeStruct((NP, 1), jnp.float32),
        ],
    )(degt, x0p)


CD = 128
NKD = 50176 // CD


def _spmm_body(t_tab, gidx, sidx, s2,
               giv, siv, siv2, gbuf, zb, accum,
               sem_i0, sem_i1, sem_g0, sem_g1, sem_s0, sem_s1):
    c = lax.axis_index("c")
    s = lax.axis_index("s")
    sem_i = (sem_i0, sem_i1)
    sem_g = (sem_g0, sem_g1)
    sem_s = (sem_s0, sem_s1)
    def zloop(i, _):
        for q in range(4):
            zb[i, pl.ds(q * 16, 16)] = jnp.zeros((16,), jnp.float32)
        return 0
    lax.fori_loop(0, 112, zloop, 0)
    def zcopy(i, _):
        pltpu.sync_copy(zb, accum.at[pl.ds(s * 1568 + i * 112, 112), :])
        return 0
    lax.fori_loop(0, 14, zcopy, 0)
    plsc.subcore_barrier()

    wbase = s * (EP // NS)

    def idx_cp(k, b):
        base = c * EP + wbase + k * CD
        return (pltpu.make_async_copy(gidx.at[pl.ds(base, CD)], giv.at[b, 0],
                                      sem_i[b]),
                pltpu.make_async_copy(sidx.at[pl.ds(base, CD)], siv.at[b],
                                      sem_i[b]))

    def gather_cps(b):
        return [pltpu.make_async_copy(
            t_tab.at[giv.at[b, j]],
            gbuf.at[b, pl.ds(j * 128, 128), :], sem_g[b])
            for j in range(CD // 128)]

    def scatter_cps(b):
        return [pltpu.make_async_copy(
            gbuf.at[b, pl.ds(j * 128, 128), :],
            accum.at[siv2.at[b, j]], sem_s[b])
            for j in range(CD // 128)]

    for cp in idx_cp(0, 0):
        cp.start()
    for cp in idx_cp(1, 1):
        cp.start()

    def half(k2, k, b):
        nb = 1 - b
        for cp in idx_cp(k, b):
            cp.wait()
        for q in range(CD // 16):
            siv2[b, q // 8, pl.ds((q % 8) * 16, 16)] = siv[b, pl.ds(q * 16, 16)]
        for cp in gather_cps(b):
            cp.start()
        @pl.when(k >= 1)
        def _():
            for cp in scatter_cps(nb):
                cp.wait()
        for cp in gather_cps(b):
            cp.wait()
        for cp in scatter_cps(b):
            cp.start(add=True)
        @pl.when(k2 < (NKD // 2) - 1)
        def _():
            for cp in idx_cp(k + 2, b):
                cp.start()

    def body(k2, _):
        half(k2, 2 * k2, 0)
        half(k2, 2 * k2 + 1, 1)
        return 0

    lax.fori_loop(0, NKD // 2, body, 0)
    for cp in scatter_cps(1):
        cp.wait()
    plsc.subcore_barrier()
    pltpu.sync_copy(accum.at[pl.ds(s * 1568, 1568), :],
                    s2.at[c, pl.ds(s * 1568, 1568), :])


def _spmm(t_tab, gidx, sidx):
    f32 = jnp.float32
    return pl.kernel(
        _spmm_body,
        out_type=[jax.ShapeDtypeStruct((2, APAD, D_EMB), f32)],
        mesh=_mesh,
        compiler_params=pltpu.CompilerParams(
            needs_layout_passes=False, use_tc_tiling_on_sc=False),
        scratch_types=[
            pltpu.VMEM((2, CD // 128, 128), jnp.int32),
            pltpu.VMEM((2, CD), jnp.int32),
            pltpu.VMEM((2, CD // 128, 128), jnp.int32),
            pltpu.VMEM((2, CD, D_EMB), f32),
            pltpu.VMEM((112, D_EMB), f32),
            pltpu.VMEM_SHARED((APAD, D_EMB), f32),
            pltpu.SemaphoreType.DMA,
            pltpu.SemaphoreType.DMA,
            pltpu.SemaphoreType.DMA,
            pltpu.SemaphoreType.DMA,
            pltpu.SemaphoreType.DMA,
            pltpu.SemaphoreType.DMA,
        ],
    )(t_tab, gidx, sidx)[0]


def _layer_body(final, acc_ref, s_ref, r_ref, accn_ref, tn_ref):
    r = r_ref[...]
    h = s_ref[...] * r
    acc = acc_ref[...] + h
    if final:
        accn_ref[...] = acc * 0.25
    else:
        accn_ref[...] = acc
    tn_ref[...] = h * r


def _layer_update(acc, s, r, final):
    return pl.pallas_call(
        functools.partial(_layer_body, final),
        grid=(NP // ROWB,),
        in_specs=[
            pl.BlockSpec((ROWB, D_EMB), lambda i: (i, 0)),
            pl.BlockSpec((ROWB, D_EMB), lambda i: (i, 0)),
            pl.BlockSpec((ROWB, 1), lambda i: (i, 0)),
        ],
        out_specs=[
            pl.BlockSpec((ROWB, D_EMB), lambda i: (i, 0)),
            pl.BlockSpec((ROWB, D_EMB), lambda i: (i, 0)),
        ],
        out_shape=[
            jax.ShapeDtypeStruct((NP, D_EMB), jnp.float32),
            jax.ShapeDtypeStruct((NP, D_EMB), jnp.float32),
        ],
    )(acc, s, r)


def kernel(user, food, edge_index, Wu, bu, Wf, bf, Wq, Wk, pool_w,
           user_table, item_table):
    f32 = jnp.float32
    src = edge_index[0].astype(jnp.int32)
    dst = edge_index[1].astype(jnp.int32)
    srcg = jnp.pad(src, (0, EP - E))
    dstg = jnp.pad(dst, (0, EP - E), constant_values=NUM_USERS)

    alpha = jax.nn.softmax(pool_w)
    c0 = (alpha[0] > 0.5).astype(f32)
    c1 = (alpha[0] + alpha[1] > 0.5).astype(f32)
    c0v = jnp.full((16,), c0, f32)
    c1v = jnp.full((16,), c1, f32)

    x = jnp.concatenate([user, food], axis=0)
    qk = _make_qk(x, Wu, bu, Wf, bf, Wq, Wk)

    gidx, sidx, degp = _edge_mask(qk, srcg, dstg, c0v, c1v)

    degt = degp.reshape(2, NP).T
    x0 = jnp.concatenate(
        [user_table, item_table, jnp.zeros((NP - NR, D_EMB), f32)], axis=0)
    t_tab, r = _make_scale(degt, x0)

    acc = x0
    for layer in range(3):
        s2 = _spmm(t_tab, gidx, sidx)
        s_full = jnp.concatenate(
            [s2[0, :NUM_USERS], s2[1, :NUM_FOODS],
             jnp.zeros((NP - NR, D_EMB), f32)], axis=0)
        acc, t_tab = _layer_update(acc, s_full, r, final=(layer == 2))

    users_final = acc[:NUM_USERS]
    items_final = acc[NUM_USERS:NR]
    return users_final, items_final, user_table, item_table


# speedup vs baseline: 11.0333x; 1.0449x over previous
import functools

import jax
import jax.numpy as jnp
from jax import lax
from jax.experimental import pallas as pl
from jax.experimental.pallas import tpu as pltpu
from jax.experimental.pallas import tpu_sc as plsc

NUM_USERS = 25000
NUM_FOODS = 25000
NR = NUM_USERS + NUM_FOODS
D_IN = 128
D_EMB = 64
N_HEADS = 4
HEAD_DIM = 16
THRESH = 0.3
E = 800000

NC = 2
NS = 16
NP = 49 * 1024
APAD = 25088
EP = 2 * 16 * 25088
E_PER_SC = EP // NC
CH = 128
ROWB = 1024

_mesh = plsc.VectorSubcoreMesh(
    core_axis_name="c", subcore_axis_name="s", num_cores=NC, num_subcores=NS)


def _proj_kernel_body(nblk_user, x_ref, w_ref, b_ref, wqk_ref, o_ref):
    pid = pl.program_id(0)
    is_user = pid < nblk_user
    w = jnp.where(is_user, w_ref[0], w_ref[1])
    b = jnp.where(is_user, b_ref[0], b_ref[1])
    wqk = jnp.where(is_user, wqk_ref[0], wqk_ref[1])
    emb = jnp.maximum(
        jnp.dot(x_ref[...], w, preferred_element_type=jnp.float32) + b[None, :],
        0.0)
    y = jnp.dot(emb, wqk, preferred_element_type=jnp.float32)
    lane = lax.broadcasted_iota(jnp.int32, y.shape, 1)
    scale = jnp.zeros_like(y)
    for h in range(N_HEADS):
        m = (lane >= h * HEAD_DIM) & (lane < (h + 1) * HEAD_DIM)
        n2 = jnp.sum(jnp.where(m, y * y, 0.0), axis=1, keepdims=True)
        scale = scale + jnp.where(m, lax.rsqrt(n2 + 1e-16), 0.0)
    o_ref[...] = y * scale


def _make_qk(x, Wu, bu, Wf, bf, Wq, Wk):
    nb = NR // 1000
    wstk = jnp.stack([Wu, Wf])
    bstk = jnp.stack([bu, bf])
    qstk = jnp.stack([Wq, Wk])
    return pl.pallas_call(
        functools.partial(_proj_kernel_body, NUM_USERS // 1000),
        grid=(nb,),
        in_specs=[
            pl.BlockSpec((1000, D_IN), lambda i: (i, 0)),
            pl.BlockSpec((2, D_IN, D_EMB), lambda i: (0, 0, 0)),
            pl.BlockSpec((2, D_EMB), lambda i: (0, 0)),
            pl.BlockSpec((2, D_EMB, D_EMB), lambda i: (0, 0, 0)),
        ],
        out_specs=pl.BlockSpec((1000, D_EMB), lambda i: (i, 0)),
        out_shape=jax.ShapeDtypeStruct((NR, D_EMB), jnp.float32),
    )(x, wstk, bstk, qstk)


CB = 128
NKB = 25088 // CB


def _edge_mask_body(qk, srcg, dstg, c0v, c1v,
                    gidx, sidx, degp,
                    sv, dv, gfb, gub, sub, sfb, dub, ddb,
                    ubuf, fbuf, onesb, zb, consts, pT, degacc,
                    sem_i0, sem_i1, sem_g0, sem_g1, sem_o0, sem_o1):
    c = lax.axis_index("c")
    s = lax.axis_index("s")
    sem_i = (sem_i0, sem_i1)
    sem_g = (sem_g0, sem_g1)
    sem_o = (sem_o0, sem_o1)
    pltpu.sync_copy(c0v, consts.at[0])
    pltpu.sync_copy(c1v, consts.at[1])

    def zloop(i, _):
        zb[pl.ds(i * 16, 16)] = jnp.zeros((16,), jnp.float32)
        return 0
    lax.fori_loop(0, 784 // 16, zloop, 0)
    def zcopy(i, _):
        pltpu.sync_copy(zb, degacc.at[pl.ds(s * 3136 + i * 784, 784)])
        return 0
    lax.fori_loop(0, 4, zcopy, 0)
    def oloop(i, _):
        onesb[pl.ds(i * 16, 16)] = jnp.ones((16,), jnp.float32)
        return 0
    lax.fori_loop(0, 128 // 16, oloop, 0)
    plsc.subcore_barrier()

    c0 = consts[0]
    c1 = consts[1]
    wbase = (c * NS + s) * 25088

    def idx_cp(k, b):
        base = wbase + k * CB
        return (pltpu.make_async_copy(srcg.at[pl.ds(base, CB)], sv.at[b],
                                      sem_i[b]),
                pltpu.make_async_copy(dstg.at[pl.ds(base, CB)], dv.at[b],
                                      sem_i[b]))

    def gather_cps(b):
        return (pltpu.make_async_copy(qk.at[sv.at[b]], ubuf.at[b], sem_g[b]),
                pltpu.make_async_copy(qk.at[dv.at[b]], fbuf.at[b], sem_g[b]))

    def out_cps(k, b):
        base = wbase + k * CB
        return (
            pltpu.make_async_copy(gfb.at[b], gidx.at[pl.ds(base, CB)],
                                  sem_o[b]),
            pltpu.make_async_copy(gub.at[b], gidx.at[pl.ds(EP + base, CB)],
                                  sem_o[b]),
            pltpu.make_async_copy(sub.at[b], sidx.at[pl.ds(base, CB)],
                                  sem_o[b]),
            pltpu.make_async_copy(sfb.at[b], sidx.at[pl.ds(EP + base, CB)],
                                  sem_o[b]),
        )

    for cp in idx_cp(0, 0):
        cp.start()
    for cp in idx_cp(0, 0):
        cp.wait()
    for cp in gather_cps(0):
        cp.start()
    for cp in idx_cp(1, 1):
        cp.start()

    def compute_chunk(k, b):
        base = wbase + k * CB
        lane = lax.iota(jnp.int32, 16)
        lane129 = lane * 129

        def edge(e, _):
            for u in range(2):
                ee = e * 2 + u
                for q in range(4):
                    uv = ubuf[b, ee, pl.ds(q * 16, 16)]
                    fv = fbuf[b, ee, pl.ds(q * 16, 16)]
                    plsc.store_scatter(pT, [q * 2064 + lane129 + ee], uv * fv)
            return 0

        lax.fori_loop(0, CB // 2, edge, 0)

        def group(g, _):
            rows = g * 16 + lax.iota(jnp.int32, 16)
            acc = jnp.zeros((16,), jnp.float32)
            for d in range(D_EMB):
                acc = acc + pT[pl.ds(d * 129 + g * 16, 16)]
            eid = base + rows
            w = jnp.where(acc * 0.25 > THRESH, c1, c0)
            act = (w > 0.5) & (eid < E)
            sval = sv[b, pl.ds(g * 16, 16)]
            dval = dv[b, pl.ds(g * 16, 16)]
            tpad = NR + (eid & 127)
            apad = NUM_USERS + (eid & 63)
            gfb[b, pl.ds(g * 16, 16)] = jnp.where(act, dval, tpad)
            gub[b, pl.ds(g * 16, 16)] = jnp.where(act, sval, tpad)
            sub[b, pl.ds(g * 16, 16)] = jnp.where(act, sval, apad)
            sfb[b, pl.ds(g * 16, 16)] = jnp.where(act, dval - NUM_USERS, apad)
            dub[pl.ds(g * 16, 16)] = jnp.where(act, sval, NR + (eid & 127))
            ddb[pl.ds(g * 16, 16)] = jnp.where(act, dval, NR + (eid & 127))
            return 0

        lax.fori_loop(0, CB // 16, group, 0)

    def half(k2, k, b):
        nb = 1 - b
        for cp in gather_cps(b):
            cp.wait()
        @pl.when(k < NKB - 1)
        def _():
            for cp in idx_cp(k + 1, nb):
                cp.wait()
            for cp in gather_cps(nb):
                cp.start()
        compute_chunk(k, b)
        pltpu.sync_copy(onesb, degacc.at[dub], add=True)
        pltpu.sync_copy(onesb, degacc.at[ddb], add=True)
        @pl.when(k2 >= 1)
        def _():
            for cp in out_cps(k - 2, b):
                cp.wait()
        for cp in out_cps(k, b):
            cp.start()
        @pl.when(k2 < (NKB // 2) - 1)
        def _():
            for cp in idx_cp(k + 2, b):
                cp.start()

    def body(k2, _):
        half(k2, 2 * k2, 0)
        half(k2, 2 * k2 + 1, 1)
        return 0

    lax.fori_loop(0, NKB // 2, body, 0)
    for cp in out_cps(NKB - 2, 0):
        cp.wait()
    for cp in out_cps(NKB - 1, 1):
        cp.wait()
    plsc.subcore_barrier()
    pltpu.sync_copy(degacc.at[pl.ds(s * 3136, 3136)],
                    degp.at[pl.ds(c * NP + s * 3136, 3136)])


def _edge_mask(qk, srcg, dstg, c0v, c1v):
    f32 = jnp.float32
    i32 = jnp.int32
    return pl.kernel(
        _edge_mask_body,
        out_type=[
            jax.ShapeDtypeStruct((2 * EP,), i32),
            jax.ShapeDtypeStruct((2 * EP,), i32),
            jax.ShapeDtypeStruct((2 * NP,), f32),
        ],
        mesh=_mesh,
        compiler_params=pltpu.CompilerParams(
            needs_layout_passes=False, use_tc_tiling_on_sc=False),
        scratch_types=[
            pltpu.VMEM((2, CB), i32),
            pltpu.VMEM((2, CB), i32),
            pltpu.VMEM((2, CB), i32),
            pltpu.VMEM((2, CB), i32),
            pltpu.VMEM((2, CB), i32),
            pltpu.VMEM((2, CB), i32),
            pltpu.VMEM((CB,), i32),
            pltpu.VMEM((CB,), i32),
            pltpu.VMEM((2, CB, D_EMB), f32),
            pltpu.VMEM((2, CB, D_EMB), f32),
            pltpu.VMEM((CB,), f32),
            pltpu.VMEM((784,), f32),
            pltpu.VMEM((2, 16), f32),
            pltpu.VMEM((8256,), f32),
            pltpu.VMEM_SHARED((NP,), f32),
            pltpu.SemaphoreType.DMA,
            pltpu.SemaphoreType.DMA,
            pltpu.SemaphoreType.DMA,
            pltpu.SemaphoreType.DMA,
            pltpu.SemaphoreType.DMA,
            pltpu.SemaphoreType.DMA,
        ],
    )(qk, srcg, dstg, c0v, c1v)


def _scale_body(degt_ref, x0_ref, t0_ref, r_ref):
    deg = degt_ref[:, 0:1] + degt_ref[:, 1:2]
    r = lax.rsqrt(jnp.maximum(deg, 0.5))
    r_ref[...] = r
    t0_ref[...] = x0_ref[...] * r


def _make_scale(degt, x0p):
    return pl.pallas_call(
        _scale_body,
        grid=(NP // ROWB,),
        in_specs=[
            pl.BlockSpec((ROWB, 2), lambda i: (i, 0)),
            pl.BlockSpec((ROWB, D_EMB), lambda i: (i, 0)),
        ],
        out_specs=[
            pl.BlockSpec((ROWB, D_EMB), lambda i: (i, 0)),
            pl.BlockSpec((ROWB, 1), lambda i: (i, 0)),
        ],
        out_shape=[
            jax.ShapeDtypeStruct((NP, D_EMB), jnp.float32),
            jax.ShapeDtypeStruct((NP, 1), jnp.float32),
        ],
    )(degt, x0p)


CD = 128
NKD = 50176 // CD


def _spmm_body(t_tab, gidx, sidx, s2,
               giv, siv, siv2, gbuf, zb, accum,
               sem_i0, sem_i1, sem_g0, sem_g1, sem_s0, sem_s1):
    c = lax.axis_index("c")
    s = lax.axis_index("s")
    sem_i = (sem_i0, sem_i1)
    sem_g = (sem_g0, sem_g1)
    sem_s = (sem_s0, sem_s1)
    def zloop(i, _):
        for q in range(4):
            zb[i, pl.ds(q * 16, 16)] = jnp.zeros((16,), jnp.float32)
        return 0
    lax.fori_loop(0, 112, zloop, 0)
    def zcopy(i, _):
        pltpu.sync_copy(zb, accum.at[pl.ds(s * 1568 + i * 112, 112), :])
        return 0
    lax.fori_loop(0, 14, zcopy, 0)
    plsc.subcore_barrier()

    wbase = s * (EP // NS)

    def idx_cp(k, b):
        base = c * EP + wbase + k * CD
        return (pltpu.make_async_copy(gidx.at[pl.ds(base, CD)], giv.at[b, 0],
                                      sem_i[b]),
                pltpu.make_async_copy(sidx.at[pl.ds(base, CD)], siv.at[b],
                                      sem_i[b]))

    def gather_cps(b):
        return [pltpu.make_async_copy(
            t_tab.at[giv.at[b, j]],
            gbuf.at[b, pl.ds(j * 128, 128), :], sem_g[b])
            for j in range(CD // 128)]

    def scatter_cps(b):
        return [pltpu.make_async_copy(
            gbuf.at[b, pl.ds(j * 128, 128), :],
            accum.at[siv2.at[b, j]], sem_s[b])
            for j in range(CD // 128)]

    for cp in idx_cp(0, 0):
        cp.start()
    for cp in idx_cp(1, 1):
        cp.start()

    def half(k2, k, b):
        nb = 1 - b
        for cp in idx_cp(k, b):
            cp.wait()
        for q in range(CD // 16):
            siv2[b, q // 8, pl.ds((q % 8) * 16, 16)] = siv[b, pl.ds(q * 16, 16)]
        for cp in gather_cps(b):
            cp.start()
        @pl.when(k >= 1)
        def _():
            for cp in scatter_cps(nb):
                cp.wait()
        for cp in gather_cps(b):
            cp.wait()
        for cp in scatter_cps(b):
            cp.start(add=True)
        @pl.when(k2 < (NKD // 2) - 1)
        def _():
            for cp in idx_cp(k + 2, b):
                cp.start()

    def body(k2, _):
        half(k2, 2 * k2, 0)
        half(k2, 2 * k2 + 1, 1)
        return 0

    lax.fori_loop(0, NKD // 2, body, 0)
    for cp in scatter_cps(1):
        cp.wait()
    plsc.subcore_barrier()
    pltpu.sync_copy(accum.at[pl.ds(s * 1568, 1568), :],
                    s2.at[c, pl.ds(s * 1568, 1568), :])


def _spmm(t_tab, gidx, sidx):
    f32 = jnp.float32
    return pl.kernel(
        _spmm_body,
        out_type=[jax.ShapeDtypeStruct((2, APAD, D_EMB), f32)],
        mesh=_mesh,
        compiler_params=pltpu.CompilerParams(
            needs_layout_passes=False, use_tc_tiling_on_sc=False),
        scratch_types=[
            pltpu.VMEM((2, CD // 128, 128), jnp.int32),
            pltpu.VMEM((2, CD), jnp.int32),
            pltpu.VMEM((2, CD // 128, 128), jnp.int32),
            pltpu.VMEM((2, CD, D_EMB), f32),
            pltpu.VMEM((112, D_EMB), f32),
            pltpu.VMEM_SHARED((APAD, D_EMB), f32),
            pltpu.SemaphoreType.DMA,
            pltpu.SemaphoreType.DMA,
            pltpu.SemaphoreType.DMA,
            pltpu.SemaphoreType.DMA,
            pltpu.SemaphoreType.DMA,
            pltpu.SemaphoreType.DMA,
        ],
    )(t_tab, gidx, sidx)[0]


def _layer_body(final, acc_ref, s_ref, r_ref, accn_ref, tn_ref):
    r = r_ref[...]
    h = s_ref[...] * r
    acc = acc_ref[...] + h
    if final:
        accn_ref[...] = acc * 0.25
    else:
        accn_ref[...] = acc
    tn_ref[...] = h * r


def _layer_update(acc, s, r, final):
    return pl.pallas_call(
        functools.partial(_layer_body, final),
        grid=(NP // ROWB,),
        in_specs=[
            pl.BlockSpec((ROWB, D_EMB), lambda i: (i, 0)),
            pl.BlockSpec((ROWB, D_EMB), lambda i: (i, 0)),
            pl.BlockSpec((ROWB, 1), lambda i: (i, 0)),
        ],
        out_specs=[
            pl.BlockSpec((ROWB, D_EMB), lambda i: (i, 0)),
            pl.BlockSpec((ROWB, D_EMB), lambda i: (i, 0)),
        ],
        out_shape=[
            jax.ShapeDtypeStruct((NP, D_EMB), jnp.float32),
            jax.ShapeDtypeStruct((NP, D_EMB), jnp.float32),
        ],
    )(acc, s, r)


def kernel(user, food, edge_index, Wu, bu, Wf, bf, Wq, Wk, pool_w,
           user_table, item_table):
    f32 = jnp.float32
    src = edge_index[0].astype(jnp.int32)
    dst = edge_index[1].astype(jnp.int32)
    srcg = jnp.pad(src, (0, EP - E))
    dstg = jnp.pad(dst, (0, EP - E), constant_values=NUM_USERS)

    alpha = jax.nn.softmax(pool_w)
    c0 = (alpha[0] > 0.5).astype(f32)
    c1 = (alpha[0] + alpha[1] > 0.5).astype(f32)
    c0v = jnp.full((16,), c0, f32)
    c1v = jnp.full((16,), c1, f32)

    x = jnp.concatenate([user, food], axis=0)
    qk = _make_qk(x, Wu, bu, Wf, bf, Wq, Wk)

    gidx, sidx, degp = _edge_mask(qk, srcg, dstg, c0v, c1v)

    degt = degp.reshape(2, NP).T
    x0 = jnp.concatenate(
        [user_table, item_table, jnp.zeros((NP - NR, D_EMB), f32)], axis=0)
    t_tab, r = _make_scale(degt, x0)

    acc = x0
    for layer in range(3):
        s2 = _spmm(t_tab, gidx, sidx)
        s_full = jnp.concatenate(
            [s2[0, :NUM_USERS], s2[1, :NUM_FOODS],
             jnp.zeros((NP - NR, D_EMB), f32)], axis=0)
        acc, t_tab = _layer_update(acc, s_full, r, final=(layer == 2))

    users_final = acc[:NUM_USERS]
    items_final = acc[NUM_USERS:NR]
    return users_final, items_final, user_table, item_table
